# baseline (speedup 1.0000x reference)
import functools

import jax
import jax.numpy as jnp
from jax import lax
from jax.experimental import pallas as pl
from jax.experimental.pallas import tpu as pltpu
from jax.experimental.pallas import tpu_sc as plsc

NC = 2
NS = 16
L = 16


def _mm_scale_body(x_ref, w_ref, dinv_ref, o_ref):
    hw = jnp.dot(x_ref[...], w_ref[...], preferred_element_type=jnp.float32)
    o_ref[...] = dinv_ref[...] * hw


def _mm_scale(x, W, dinv, block_m=2000):
    M, K = x.shape
    _, N = W.shape
    return pl.pallas_call(
        _mm_scale_body,
        grid=(M // block_m,),
        in_specs=[
            pl.BlockSpec((block_m, K), lambda i: (i, 0)),
            pl.BlockSpec((K, N), lambda i: (0, 0)),
            pl.BlockSpec((block_m, 1), lambda i: (i, 0)),
        ],
        out_specs=pl.BlockSpec((block_m, N), lambda i: (i, 0)),
        out_shape=jax.ShapeDtypeStruct((M, N), jnp.float32),
    )(x, W, dinv)


def _epi_body(acc_ref, hs_ref, dinv_ref, nv_ref, b_ref, ws_ref, h_ref, s_ref):
    h = dinv_ref[...] * (acc_ref[...] + hs_ref[...]) + b_ref[...]
    h = jnp.maximum(h, 0.0) * nv_ref[...]
    h_ref[...] = h
    s_ref[...] = jnp.dot(h, ws_ref[...], preferred_element_type=jnp.float32)


def _epilogue(acc, hs, dinv, nv, b, ws, block_m=2000):
    M, N = acc.shape
    return pl.pallas_call(
        _epi_body,
        grid=(M // block_m,),
        in_specs=[
            pl.BlockSpec((block_m, N), lambda i: (i, 0)),
            pl.BlockSpec((block_m, N), lambda i: (i, 0)),
            pl.BlockSpec((block_m, 1), lambda i: (i, 0)),
            pl.BlockSpec((block_m, 1), lambda i: (i, 0)),
            pl.BlockSpec((1, N), lambda i: (0, 0)),
            pl.BlockSpec((N, 1), lambda i: (0, 0)),
        ],
        out_specs=(pl.BlockSpec((block_m, N), lambda i: (i, 0)),
                   pl.BlockSpec((block_m, 1), lambda i: (i, 0))),
        out_shape=(jax.ShapeDtypeStruct((M, N), jnp.float32),
                   jax.ShapeDtypeStruct((M, 1), jnp.float32)),
    )(acc, hs, dinv, nv, b.reshape(1, -1), ws)


def _head_body(h_ref, nv_ref, wl1_ref, bl1_ref, wl2_ref, bl2_ref, o_ref):
    h = h_ref[...]
    nv = nv_ref[...]
    gmax = jnp.max(jnp.where(nv > 0, h, -jnp.inf), axis=0, keepdims=True)
    gsum = jnp.sum(h, axis=0, keepdims=True)
    cnt = jnp.maximum(jnp.sum(nv), 1.0)
    g = jnp.concatenate([gmax, gsum / cnt], axis=1)
    z = jnp.maximum(jnp.dot(g, wl1_ref[...], preferred_element_type=jnp.float32)
                    + bl1_ref[...], 0.0)
    logits = jnp.dot(z, wl2_ref[...], preferred_element_type=jnp.float32) + bl2_ref[...]
    o_ref[...] = jax.nn.log_softmax(logits, axis=-1)


def _head(h, nv_f32, Wl1, bl1, Wl2, bl2):
    M, _ = h.shape
    return pl.pallas_call(
        _head_body,
        out_shape=jax.ShapeDtypeStruct((1, Wl2.shape[1]), jnp.float32),
    )(h, nv_f32.reshape(M, 1), Wl1, bl1.reshape(1, -1), Wl2, bl2.reshape(1, -1))


_SEG = 2000


@functools.cache
def _make_agg(N, E, F):
    CH = 64 if F > 128 else 128
    NP = 2 if F >= 512 else 1
    P = NC * NS * NP
    R = ((N + P - 1) // P + 7) // 8 * 8
    CAP = _SEG + CH
    NSEG = (E + _SEG - 1) // _SEG
    assert E % _SEG == 0
    mesh = plsc.VectorSubcoreMesh(core_axis_name="c", subcore_axis_name="s")

    @functools.partial(
        pl.kernel,
        out_type=jax.ShapeDtypeStruct((P * R, F), jnp.float32),
        mesh=mesh,
        compiler_params=pltpu.CompilerParams(needs_layout_passes=False),
        scratch_types=[
            pltpu.VMEM((_SEG,), jnp.int32),
            pltpu.VMEM((_SEG,), jnp.int32),
            pltpu.VMEM((CAP,), jnp.int32),
            pltpu.VMEM((CAP,), jnp.int32),
            pltpu.VMEM((CH,), jnp.int32),
            pltpu.VMEM((CH, F), jnp.float32),
            pltpu.VMEM((R + 8, F), jnp.float32),
            pltpu.SemaphoreType.DMA,
        ],
    )
    def agg(hs_hbm, src_hbm, dst_hbm, zrows_hbm, out_hbm,
            src_v, dst_v, csrc, cloc, gidx, gbuf, acc, gsem):
        c = lax.axis_index("c")
        s = lax.axis_index("s")
        for q in range(NP):
            pt = (c * NS + s) * NP + q
            lo = pt * R
            hi = jnp.minimum(lo + R, N)

            off = 0
            while off < R:
                n = min(128, R - off)
                pltpu.sync_copy(zrows_hbm.at[pl.ds(0, n)],
                                acc.at[pl.ds(off, n)])
                off += n

            def seg_body(g, _):
                pltpu.sync_copy(src_hbm.at[pl.ds(g * _SEG, _SEG)], src_v)
                pltpu.sync_copy(dst_hbm.at[pl.ds(g * _SEG, _SEG)], dst_v)

                def cbody(i, m):
                    s16 = src_v[pl.ds(i * L, L)]
                    d16 = dst_v[pl.ds(i * L, L)]
                    inb = (d16 >= lo) & (d16 < hi)
                    inc = plsc.cumsum(inb.astype(jnp.int32))
                    pos = m + inc - inb.astype(jnp.int32)
                    plsc.store_scatter(csrc, [pos], s16, mask=inb)
                    plsc.store_scatter(cloc, [pos], d16 - lo, mask=inb)
                    return m + inc[L - 1]

                m = lax.fori_loop(0, _SEG // L, cbody, jnp.int32(0))

                for t in range(CH // L):
                    csrc[pl.ds(m + t * L, L)] = jnp.zeros((L,), jnp.int32)
                    cloc[pl.ds(m + t * L, L)] = jnp.full((L,), R, jnp.int32)

                nch = (m + CH - 1) // CH

                def chunk_body(j, _):
                    base = j * CH
                    for k in range(CH // L):
                        gidx[pl.ds(k * L, L)] = csrc[pl.ds(base + k * L, L)]
                    pltpu.async_copy(hs_hbm.at[gidx], gbuf, gsem).wait()

                    def row_body(i, _):
                        dl16 = cloc[pl.ds(base + i * L, L)]
                        for t in range(L):
                            dl = dl16[t]
                            for k in range(F // L):
                                sl = pl.ds(k * L, L)
                                acc[dl, sl] = acc[dl, sl] + gbuf[i * L + t, sl]
                        return 0

                    lax.fori_loop(0, CH // L, row_body, 0)
                    return 0

                lax.fori_loop(0, nch, chunk_body, 0)
                return 0

            lax.fori_loop(0, NSEG, seg_body, 0)

            off = 0
            while off < R:
                n = min(256, R - off)
                pltpu.sync_copy(acc.at[pl.ds(off, n)],
                                out_hbm.at[pl.ds(lo + off, n)])
                off += n

    return agg, P * R


def _aggregate(hs, src, dst_masked):
    N, F = hs.shape
    E = src.shape[0]
    zrows = jnp.zeros((128, F), jnp.float32)
    agg, NPAD = _make_agg(N, E, F)
    outp = agg(hs, src, dst_masked, zrows)
    return lax.slice_in_dim(outp, 0, N)


def _kmis(score, src, dst, N, node_valid, edge_valid):
    s = score.reshape(-1)
    s_eff = jnp.where(node_valid, s, -jnp.inf)
    perm = jnp.argsort(-s_eff)
    rank = jnp.zeros((N,), jnp.int32).at[perm].set(jnp.arange(N, dtype=jnp.int32))
    ss = jnp.concatenate([src, dst])
    dd = jnp.concatenate([dst, src])
    em = jnp.concatenate([edge_valid, edge_valid])
    BIG = jnp.int32(N)

    def cond(state):
        _, mask = state
        return jnp.any(mask)

    def body(state):
        mis, mask = state
        r = jnp.where(mask, rank, BIG)
        nmin = jnp.full((N,), BIG, jnp.int32).at[dd].min(jnp.where(em, r[ss], BIG))
        local = mask & (r <= nmin)
        mis = mis | local
        nb = jnp.zeros((N,), jnp.int32).at[dd].max(
            jnp.where(em, local[ss].astype(jnp.int32), 0)) > 0
        mask = mask & (~local) & (~nb)
        return mis, mask

    mis, _ = lax.while_loop(cond, body, (jnp.zeros((N,), bool), node_valid))
    r_mis = jnp.where(mis, rank, BIG)
    cand = jnp.full((N,), BIG, jnp.int32).at[dd].min(jnp.where(em, r_mis[ss], BIG))
    cand = jnp.minimum(cand, r_mis)
    cluster_node = perm[jnp.clip(cand, 0, N - 1)]
    Nc = jnp.sum(mis).astype(jnp.int32)
    new_id = jnp.where(mis, jnp.cumsum(mis.astype(jnp.int32)) - 1, 0)
    cluster = new_id[cluster_node]
    cu = cluster[src]
    cv = cluster[dst]
    keep = (cu != cv) & edge_valid
    SENT = jnp.int32(jnp.iinfo(jnp.int32).max)
    key = jnp.sort(jnp.where(keep, cu * Nc + cv, SENT))
    uniq = (key < SENT) & jnp.concatenate(
        [jnp.ones((1,), bool), key[1:] != key[:-1]])
    den = jnp.maximum(Nc, 1)
    new_src = jnp.where(uniq, key // den, 0).astype(jnp.int32)
    new_dst = jnp.where(uniq, key % den, 0).astype(jnp.int32)
    return mis, new_id, Nc, new_src, new_dst, uniq


def _dinv_of(deg):
    return jnp.where(deg > 0, lax.rsqrt(deg), 0.0)[:, None]


def kernel(x, edge_index, batch, W1, b1, ws1, bs1, W2, b2, ws2, bs2,
           W3, b3, Wl1, bl1, Wl2, bl2):
    src = edge_index[0]
    dst = edge_index[1]
    N = x.shape[0]
    E = src.shape[0]
    ones_n = jnp.ones((N,), bool)
    ones_e = jnp.ones((E,), bool)
    ones_col = jnp.ones((N, 1), jnp.float32)
    iota_n = jnp.arange(N, dtype=jnp.int32)

    deg1 = jnp.zeros((N,), jnp.float32).at[dst].add(1.0) + 1.0
    dinv1 = _dinv_of(deg1)
    hs1 = _mm_scale(x, W1, dinv1)
    acc1 = _aggregate(hs1, src, dst)
    h, s1 = _epilogue(acc1, hs1, dinv1, ones_col, b1, ws1)
    s1 = s1 + bs1

    mis1, nid1, Nc1, src1, dst1, ev2 = _kmis(s1, src, dst, N, ones_n, ones_e)
    idx1 = jnp.where(mis1, nid1, N)
    val1 = h * s1
    h = jnp.zeros_like(val1).at[idx1].set(val1, mode="drop")
    nv2 = iota_n < Nc1
    nv2f = nv2.astype(jnp.float32)

    dstm2 = jnp.where(ev2, dst1, N)
    deg2 = (jnp.zeros((N,), jnp.float32)
            .at[dst1].add(jnp.where(ev2, 1.0, 0.0)) + nv2f)
    dinv2 = _dinv_of(deg2)
    hs2 = _mm_scale(h, W2, dinv2)
    acc2 = _aggregate(hs2, src1, dstm2)
    h, s2 = _epilogue(acc2, hs2, dinv2, nv2f[:, None], b2, ws2)
    s2 = s2 + bs2

    mis2, nid2, Nc2, src2, dst2, ev3 = _kmis(s2, src1, dst1, N, nv2, ev2)
    idx2 = jnp.where(mis2, nid2, N)
    val2 = h * s2
    h = jnp.zeros_like(val2).at[idx2].set(val2, mode="drop")
    nv3 = iota_n < Nc2
    nv3f = nv3.astype(jnp.float32)

    dstm3 = jnp.where(ev3, dst2, N)
    deg3 = (jnp.zeros((N,), jnp.float32)
            .at[dst2].add(jnp.where(ev3, 1.0, 0.0)) + nv3f)
    dinv3 = _dinv_of(deg3)
    hs3 = _mm_scale(h, W3, dinv3)
    acc3 = _aggregate(hs3, src2, dstm3)
    h, _ = _epilogue(acc3, hs3, dinv3, nv3f[:, None], b3,
                     jnp.zeros((W3.shape[1], 1), jnp.float32))

    return _head(h, nv3f, Wl1, bl1, Wl2, bl2)


# speedup vs baseline: 4.2498x; 4.2498x over previous
import functools

import jax
import jax.numpy as jnp
from jax import lax
from jax.experimental import pallas as pl
from jax.experimental.pallas import tpu as pltpu
from jax.experimental.pallas import tpu_sc as plsc

NC = 2
NS = 16
L = 16


def _mm_scale_body(x_ref, w_ref, dinv_ref, o_ref):
    hw = jnp.dot(x_ref[...], w_ref[...], preferred_element_type=jnp.float32)
    o_ref[...] = dinv_ref[...] * hw


def _mm_scale(x, W, dinv, block_m=2000):
    M, K = x.shape
    _, N = W.shape
    return pl.pallas_call(
        _mm_scale_body,
        grid=(M // block_m,),
        in_specs=[
            pl.BlockSpec((block_m, K), lambda i: (i, 0)),
            pl.BlockSpec((K, N), lambda i: (0, 0)),
            pl.BlockSpec((block_m, 1), lambda i: (i, 0)),
        ],
        out_specs=pl.BlockSpec((block_m, N), lambda i: (i, 0)),
        out_shape=jax.ShapeDtypeStruct((M, N), jnp.float32),
    )(x, W, dinv)


def _epi_body(acc_ref, hs_ref, dinv_ref, nv_ref, b_ref, ws_ref, h_ref, s_ref):
    h = dinv_ref[...] * (acc_ref[...] + hs_ref[...]) + b_ref[...]
    h = jnp.maximum(h, 0.0) * nv_ref[...]
    h_ref[...] = h
    s_ref[...] = jnp.dot(h, ws_ref[...], preferred_element_type=jnp.float32)


def _epilogue(acc, hs, dinv, nv, b, ws, block_m=2000):
    M, N = acc.shape
    return pl.pallas_call(
        _epi_body,
        grid=(M // block_m,),
        in_specs=[
            pl.BlockSpec((block_m, N), lambda i: (i, 0)),
            pl.BlockSpec((block_m, N), lambda i: (i, 0)),
            pl.BlockSpec((block_m, 1), lambda i: (i, 0)),
            pl.BlockSpec((block_m, 1), lambda i: (i, 0)),
            pl.BlockSpec((1, N), lambda i: (0, 0)),
            pl.BlockSpec((N, 1), lambda i: (0, 0)),
        ],
        out_specs=(pl.BlockSpec((block_m, N), lambda i: (i, 0)),
                   pl.BlockSpec((block_m, 1), lambda i: (i, 0))),
        out_shape=(jax.ShapeDtypeStruct((M, N), jnp.float32),
                   jax.ShapeDtypeStruct((M, 1), jnp.float32)),
    )(acc, hs, dinv, nv, b.reshape(1, -1), ws)


def _head_body(h_ref, nv_ref, wl1_ref, bl1_ref, wl2_ref, bl2_ref, o_ref):
    h = h_ref[...]
    nv = nv_ref[...]
    gmax = jnp.max(jnp.where(nv > 0, h, -jnp.inf), axis=0, keepdims=True)
    gsum = jnp.sum(h, axis=0, keepdims=True)
    cnt = jnp.maximum(jnp.sum(nv), 1.0)
    g = jnp.concatenate([gmax, gsum / cnt], axis=1)
    z = jnp.maximum(jnp.dot(g, wl1_ref[...], preferred_element_type=jnp.float32)
                    + bl1_ref[...], 0.0)
    logits = jnp.dot(z, wl2_ref[...], preferred_element_type=jnp.float32) + bl2_ref[...]
    o_ref[...] = jax.nn.log_softmax(logits, axis=-1)


def _head(h, nv_f32, Wl1, bl1, Wl2, bl2):
    M, _ = h.shape
    return pl.pallas_call(
        _head_body,
        out_shape=jax.ShapeDtypeStruct((1, Wl2.shape[1]), jnp.float32),
    )(h, nv_f32.reshape(M, 1), Wl1, bl1.reshape(1, -1), Wl2, bl2.reshape(1, -1))


_SEG = 2000


@functools.cache
def _make_agg(N, E, F):
    CH = 64 if F > 128 else 128
    NP = 2 if F >= 512 else 1
    P = NC * NS * NP
    R = ((N + P - 1) // P + 7) // 8 * 8
    CAP = _SEG + CH
    NSEG = (E + _SEG - 1) // _SEG
    assert E % _SEG == 0
    mesh = plsc.VectorSubcoreMesh(core_axis_name="c", subcore_axis_name="s")

    @functools.partial(
        pl.kernel,
        out_type=jax.ShapeDtypeStruct((P * R, F), jnp.float32),
        mesh=mesh,
        compiler_params=pltpu.CompilerParams(needs_layout_passes=False),
        scratch_types=[
            pltpu.VMEM((_SEG,), jnp.int32),
            pltpu.VMEM((_SEG,), jnp.int32),
            pltpu.VMEM((CAP,), jnp.int32),
            pltpu.VMEM((CAP,), jnp.int32),
            pltpu.VMEM((CH,), jnp.int32),
            pltpu.VMEM((CH, F), jnp.float32),
            pltpu.VMEM((R + 8, F), jnp.float32),
            pltpu.SemaphoreType.DMA,
        ],
    )
    def agg(hs_hbm, src_hbm, dst_hbm, zrows_hbm, out_hbm,
            src_v, dst_v, csrc, cloc, gidx, gbuf, acc, gsem):
        c = lax.axis_index("c")
        s = lax.axis_index("s")
        for q in range(NP):
            pt = (c * NS + s) * NP + q
            lo = pt * R
            hi = jnp.minimum(lo + R, N)

            off = 0
            while off < R:
                n = min(128, R - off)
                pltpu.sync_copy(zrows_hbm.at[pl.ds(0, n)],
                                acc.at[pl.ds(off, n)])
                off += n

            def seg_body(g, _):
                pltpu.sync_copy(src_hbm.at[pl.ds(g * _SEG, _SEG)], src_v)
                pltpu.sync_copy(dst_hbm.at[pl.ds(g * _SEG, _SEG)], dst_v)

                def cbody(i, m):
                    s16 = src_v[pl.ds(i * L, L)]
                    d16 = dst_v[pl.ds(i * L, L)]
                    inb = (d16 >= lo) & (d16 < hi)
                    inc = plsc.cumsum(inb.astype(jnp.int32))
                    pos = m + inc - inb.astype(jnp.int32)
                    plsc.store_scatter(csrc, [pos], s16, mask=inb)
                    plsc.store_scatter(cloc, [pos], d16 - lo, mask=inb)
                    return m + inc[L - 1]

                m = lax.fori_loop(0, _SEG // L, cbody, jnp.int32(0))

                for t in range(CH // L):
                    csrc[pl.ds(m + t * L, L)] = jnp.zeros((L,), jnp.int32)
                    cloc[pl.ds(m + t * L, L)] = jnp.full((L,), R, jnp.int32)

                nch = (m + CH - 1) // CH

                def chunk_body(j, _):
                    base = j * CH
                    for k in range(CH // L):
                        gidx[pl.ds(k * L, L)] = csrc[pl.ds(base + k * L, L)]
                    pltpu.async_copy(hs_hbm.at[gidx], gbuf, gsem).wait()

                    def row_body(i, _):
                        dl16 = cloc[pl.ds(base + i * L, L)]
                        for t in range(L):
                            dl = dl16[t]
                            for k in range(F // L):
                                sl = pl.ds(k * L, L)
                                acc[dl, sl] = acc[dl, sl] + gbuf[i * L + t, sl]
                        return 0

                    lax.fori_loop(0, CH // L, row_body, 0)
                    return 0

                lax.fori_loop(0, nch, chunk_body, 0)
                return 0

            lax.fori_loop(0, NSEG, seg_body, 0)

            off = 0
            while off < R:
                n = min(256, R - off)
                pltpu.sync_copy(acc.at[pl.ds(off, n)],
                                out_hbm.at[pl.ds(lo + off, n)])
                off += n

    return agg, P * R


def _aggregate(hs, src, dst_masked):
    N, F = hs.shape
    E = src.shape[0]
    zrows = jnp.zeros((128, F), jnp.float32)
    agg, NPAD = _make_agg(N, E, F)
    outp = agg(hs, src, dst_masked, zrows)
    return lax.slice_in_dim(outp, 0, N)


@functools.cache
def _make_mis(N, E2):
    NPAD = 10240
    OWN = NPAD // NS
    EPT = E2 // NS
    BIG = N
    mesh = plsc.VectorSubcoreMesh(core_axis_name="c", subcore_axis_name="s",
                                  num_cores=1)
    i32 = jnp.int32

    @functools.partial(
        pl.kernel,
        out_type=jax.ShapeDtypeStruct((NPAD,), i32),
        mesh=mesh,
        compiler_params=pltpu.CompilerParams(needs_layout_passes=False),
        scratch_types=[
            pltpu.VMEM((EPT,), i32),
            pltpu.VMEM((NPAD,), i32),
            pltpu.VMEM((NPAD,), i32),
            pltpu.VMEM((NPAD,), i32),
            pltpu.VMEM((NPAD,), i32),
            pltpu.VMEM((OWN,), i32),
            pltpu.VMEM((OWN,), i32),
            pltpu.VMEM((OWN,), i32),
            pltpu.VMEM((NS, OWN), i32),
            pltpu.VMEM((NS, L), i32),
            pltpu.VMEM((L,), i32),
            pltpu.VMEM_SHARED((NS, NPAD), i32),
            pltpu.VMEM_SHARED((NPAD,), i32),
            pltpu.VMEM_SHARED((NS, L), i32),
        ],
    )
    def mis_k(packed_hbm, perm_hbm, v_hbm, mis_hbm,
              edges_v, rank_t, mask_t, killed_t, local_t,
              rank_own, local_own, mis_own, mbuf, fbuf, vbuf,
              pub, garr, gflags):
        t = lax.axis_index("s")
        own0 = t * OWN
        iota = lax.iota(i32, L)
        ones16 = jnp.ones((L,), i32)
        zeros16 = jnp.zeros((L,), i32)

        pltpu.sync_copy(packed_hbm.at[pl.ds(t * EPT, EPT)], edges_v)
        pltpu.sync_copy(v_hbm, vbuf)
        V = vbuf[pl.ds(0, L)][0]

        def ibody(j, _):
            idx16 = j * L + iota
            mask_t[pl.ds(j * L, L)] = (idx16 < V).astype(i32)
            return 0
        lax.fori_loop(0, NPAD // L, ibody, 0)

        def i2body(j, _):
            sl = pl.ds(j * L, L)
            rank_own[sl] = jnp.full((L,), BIG, i32)
            mis_own[sl] = zeros16
            return 0
        lax.fori_loop(0, OWN // L, i2body, 0)

        pltpu.sync_copy(perm_hbm, local_t.at[pl.ds(0, N)])

        def rbody(j, _):
            p16 = local_t[pl.ds(j * L, L)]
            inown = (p16 >= own0) & (p16 < own0 + OWN)
            plsc.store_scatter(rank_own, [p16 - own0], j * L + iota, mask=inown)
            return 0
        lax.fori_loop(0, N // L, rbody, 0)

        pltpu.sync_copy(rank_own, garr.at[pl.ds(own0, OWN)])
        plsc.subcore_barrier()
        pltpu.sync_copy(garr, rank_t)
        plsc.subcore_barrier()

        def zero_killed():
            def zbody(j, _):
                killed_t[pl.ds(j * L, L)] = zeros16
                return 0
            lax.fori_loop(0, NPAD // L, zbody, 0)

        def merge_or(dst_own):
            pltpu.sync_copy(pub.at[:, pl.ds(own0, OWN)], mbuf)

            def obody(j, _):
                sl = pl.ds(j * L, L)
                acc = zeros16
                for tt in range(NS):
                    acc = acc | mbuf[tt, sl]
                dst_own[sl] = acc
                return 0
            lax.fori_loop(0, OWN // L, obody, 0)

        def loop_body(go):
            zero_killed()

            def e1body(j, _):
                p16 = edges_v[pl.ds(j * L, L)]
                d16 = p16 & 16383
                s16 = lax.shift_right_logical(p16, 14)
                rs = plsc.load_gather(rank_t, [s16])
                rd = plsc.load_gather(rank_t, [d16])
                ms = plsc.load_gather(mask_t, [s16])
                ind = (ms > 0) & (rs < rd)
                plsc.store_scatter(killed_t, [d16], ones16, mask=ind)
                return 0
            lax.fori_loop(0, EPT // L, e1body, 0)

            pltpu.sync_copy(killed_t, pub.at[t])
            plsc.subcore_barrier()
            merge_or(local_own)

            def lbody(j, _):
                sl = pl.ds(j * L, L)
                loc = jnp.where(mask_t[pl.ds(own0 + j * L, L)] > 0,
                                1 - jnp.minimum(local_own[sl], 1), 0)
                local_own[sl] = loc
                mis_own[sl] = mis_own[sl] | loc
                return 0
            lax.fori_loop(0, OWN // L, lbody, 0)

            plsc.subcore_barrier()
            pltpu.sync_copy(local_own, garr.at[pl.ds(own0, OWN)])
            plsc.subcore_barrier()
            pltpu.sync_copy(garr, local_t)

            zero_killed()

            def e2body(j, _):
                p16 = edges_v[pl.ds(j * L, L)]
                d16 = p16 & 16383
                s16 = lax.shift_right_logical(p16, 14)
                ls = plsc.load_gather(local_t, [s16])
                plsc.store_scatter(killed_t, [d16], ones16, mask=ls > 0)
                return 0
            lax.fori_loop(0, EPT // L, e2body, 0)

            plsc.subcore_barrier()
            pltpu.sync_copy(killed_t, pub.at[t])
            plsc.subcore_barrier()
            merge_or(rank_own)

            anyv = zeros16

            def ubody(j, anyv):
                sl = pl.ds(j * L, L)
                newm = (mask_t[pl.ds(own0 + j * L, L)]
                        * (1 - local_own[sl])
                        * (1 - jnp.minimum(rank_own[sl], 1)))
                local_t[pl.ds(j * L, L)] = newm
                return anyv | newm
            anyv = lax.fori_loop(0, OWN // L, ubody, anyv)

            pltpu.sync_copy(local_t.at[pl.ds(0, OWN)], garr.at[pl.ds(own0, OWN)])
            fbuf[0, pl.ds(0, L)] = jnp.minimum(anyv, 1)
            pltpu.sync_copy(fbuf.at[0], gflags.at[t])
            plsc.subcore_barrier()
            pltpu.sync_copy(garr, mask_t)
            pltpu.sync_copy(gflags, fbuf)
            plsc.subcore_barrier()

            accv = zeros16
            for tt in range(NS):
                accv = accv | fbuf[tt, pl.ds(0, L)]
            return plsc.cummax(accv)[L - 1]

        lax.while_loop(lambda go: go > 0, loop_body, jnp.int32(1))

        pltpu.sync_copy(mis_own, mis_hbm.at[pl.ds(own0, OWN)])

    return mis_k


def _kmis(score, src, dst, N, V, node_valid, edge_valid):
    s = score.reshape(-1)
    s_eff = jnp.where(node_valid, s, -jnp.inf)
    perm = jnp.argsort(-s_eff).astype(jnp.int32)
    rank = jnp.zeros((N,), jnp.int32).at[perm].set(jnp.arange(N, dtype=jnp.int32))
    ss = jnp.concatenate([src, dst])
    dd = jnp.concatenate([dst, src])
    em = jnp.concatenate([edge_valid, edge_valid])
    BIG = jnp.int32(N)

    ssm = jnp.where(em, ss, N)
    ddm = jnp.where(em, dd, N)
    packed = ssm * jnp.int32(16384) + ddm
    misI = _make_mis(N, ss.shape[0])(
        packed, perm, jnp.full((16,), V, jnp.int32))
    mis = misI[:N] > 0
    r_mis = jnp.where(mis, rank, BIG)
    cand = jnp.full((N,), BIG, jnp.int32).at[dd].min(jnp.where(em, r_mis[ss], BIG))
    cand = jnp.minimum(cand, r_mis)
    cluster_node = perm[jnp.clip(cand, 0, N - 1)]
    Nc = jnp.sum(mis).astype(jnp.int32)
    new_id = jnp.where(mis, jnp.cumsum(mis.astype(jnp.int32)) - 1, 0)
    cluster = new_id[cluster_node]
    cu = cluster[src]
    cv = cluster[dst]
    keep = (cu != cv) & edge_valid
    SENT = jnp.int32(jnp.iinfo(jnp.int32).max)
    key = jnp.sort(jnp.where(keep, cu * Nc + cv, SENT))
    uniq = (key < SENT) & jnp.concatenate(
        [jnp.ones((1,), bool), key[1:] != key[:-1]])
    den = jnp.maximum(Nc, 1)
    new_src = jnp.where(uniq, key // den, 0).astype(jnp.int32)
    new_dst = jnp.where(uniq, key % den, 0).astype(jnp.int32)
    return mis, new_id, Nc, new_src, new_dst, uniq


def _dinv_of(deg):
    return jnp.where(deg > 0, lax.rsqrt(deg), 0.0)[:, None]


def kernel(x, edge_index, batch, W1, b1, ws1, bs1, W2, b2, ws2, bs2,
           W3, b3, Wl1, bl1, Wl2, bl2):
    src = edge_index[0]
    dst = edge_index[1]
    N = x.shape[0]
    E = src.shape[0]
    ones_n = jnp.ones((N,), bool)
    ones_e = jnp.ones((E,), bool)
    ones_col = jnp.ones((N, 1), jnp.float32)
    iota_n = jnp.arange(N, dtype=jnp.int32)

    deg1 = jnp.zeros((N,), jnp.float32).at[dst].add(1.0) + 1.0
    dinv1 = _dinv_of(deg1)
    hs1 = _mm_scale(x, W1, dinv1)
    acc1 = _aggregate(hs1, src, dst)
    h, s1 = _epilogue(acc1, hs1, dinv1, ones_col, b1, ws1)
    s1 = s1 + bs1

    mis1, nid1, Nc1, src1, dst1, ev2 = _kmis(s1, src, dst, N, jnp.int32(N),
                                             ones_n, ones_e)
    idx1 = jnp.where(mis1, nid1, N)
    val1 = h * s1
    h = jnp.zeros_like(val1).at[idx1].set(val1, mode="drop")
    nv2 = iota_n < Nc1
    nv2f = nv2.astype(jnp.float32)

    dstm2 = jnp.where(ev2, dst1, N)
    deg2 = (jnp.zeros((N,), jnp.float32)
            .at[dst1].add(jnp.where(ev2, 1.0, 0.0)) + nv2f)
    dinv2 = _dinv_of(deg2)
    hs2 = _mm_scale(h, W2, dinv2)
    acc2 = _aggregate(hs2, src1, dstm2)
    h, s2 = _epilogue(acc2, hs2, dinv2, nv2f[:, None], b2, ws2)
    s2 = s2 + bs2

    mis2, nid2, Nc2, src2, dst2, ev3 = _kmis(s2, src1, dst1, N, Nc1, nv2, ev2)
    idx2 = jnp.where(mis2, nid2, N)
    val2 = h * s2
    h = jnp.zeros_like(val2).at[idx2].set(val2, mode="drop")
    nv3 = iota_n < Nc2
    nv3f = nv3.astype(jnp.float32)

    dstm3 = jnp.where(ev3, dst2, N)
    deg3 = (jnp.zeros((N,), jnp.float32)
            .at[dst2].add(jnp.where(ev3, 1.0, 0.0)) + nv3f)
    dinv3 = _dinv_of(deg3)
    hs3 = _mm_scale(h, W3, dinv3)
    acc3 = _aggregate(hs3, src2, dstm3)
    h, _ = _epilogue(acc3, hs3, dinv3, nv3f[:, None], b3,
                     jnp.zeros((W3.shape[1], 1), jnp.float32))

    return _head(h, nv3f, Wl1, bl1, Wl2, bl2)


# speedup vs baseline: 5.7945x; 1.3635x over previous
import functools

import jax
import jax.numpy as jnp
from jax import lax
from jax.experimental import pallas as pl
from jax.experimental.pallas import tpu as pltpu
from jax.experimental.pallas import tpu_sc as plsc

NC = 2
NS = 16
L = 16


def _mm_scale_body(x_ref, w_ref, dinv_ref, o_ref):
    hw = jnp.dot(x_ref[...], w_ref[...], preferred_element_type=jnp.float32)
    o_ref[...] = dinv_ref[...] * hw


def _mm_scale(x, W, dinv, block_m=2000):
    M, K = x.shape
    _, N = W.shape
    return pl.pallas_call(
        _mm_scale_body,
        grid=(M // block_m,),
        in_specs=[
            pl.BlockSpec((block_m, K), lambda i: (i, 0)),
            pl.BlockSpec((K, N), lambda i: (0, 0)),
            pl.BlockSpec((block_m, 1), lambda i: (i, 0)),
        ],
        out_specs=pl.BlockSpec((block_m, N), lambda i: (i, 0)),
        out_shape=jax.ShapeDtypeStruct((M, N), jnp.float32),
    )(x, W, dinv)


def _epi_body(acc_ref, hs_ref, dinv_ref, nv_ref, b_ref, ws_ref, h_ref, s_ref):
    h = dinv_ref[...] * (acc_ref[...] + hs_ref[...]) + b_ref[...]
    h = jnp.maximum(h, 0.0) * nv_ref[...]
    h_ref[...] = h
    s_ref[...] = jnp.dot(h, ws_ref[...], preferred_element_type=jnp.float32)


def _epilogue(acc, hs, dinv, nv, b, ws, block_m=2000):
    M, N = acc.shape
    return pl.pallas_call(
        _epi_body,
        grid=(M // block_m,),
        in_specs=[
            pl.BlockSpec((block_m, N), lambda i: (i, 0)),
            pl.BlockSpec((block_m, N), lambda i: (i, 0)),
            pl.BlockSpec((block_m, 1), lambda i: (i, 0)),
            pl.BlockSpec((block_m, 1), lambda i: (i, 0)),
            pl.BlockSpec((1, N), lambda i: (0, 0)),
            pl.BlockSpec((N, 1), lambda i: (0, 0)),
        ],
        out_specs=(pl.BlockSpec((block_m, N), lambda i: (i, 0)),
                   pl.BlockSpec((block_m, 1), lambda i: (i, 0))),
        out_shape=(jax.ShapeDtypeStruct((M, N), jnp.float32),
                   jax.ShapeDtypeStruct((M, 1), jnp.float32)),
    )(acc, hs, dinv, nv, b.reshape(1, -1), ws)


def _head_body(h_ref, nv_ref, wl1_ref, bl1_ref, wl2_ref, bl2_ref, o_ref):
    h = h_ref[...]
    nv = nv_ref[...]
    gmax = jnp.max(jnp.where(nv > 0, h, -jnp.inf), axis=0, keepdims=True)
    gsum = jnp.sum(h, axis=0, keepdims=True)
    cnt = jnp.maximum(jnp.sum(nv), 1.0)
    g = jnp.concatenate([gmax, gsum / cnt], axis=1)
    z = jnp.maximum(jnp.dot(g, wl1_ref[...], preferred_element_type=jnp.float32)
                    + bl1_ref[...], 0.0)
    logits = jnp.dot(z, wl2_ref[...], preferred_element_type=jnp.float32) + bl2_ref[...]
    o_ref[...] = jax.nn.log_softmax(logits, axis=-1)


def _head(h, nv_f32, Wl1, bl1, Wl2, bl2):
    M, _ = h.shape
    return pl.pallas_call(
        _head_body,
        out_shape=jax.ShapeDtypeStruct((1, Wl2.shape[1]), jnp.float32),
    )(h, nv_f32.reshape(M, 1), Wl1, bl1.reshape(1, -1), Wl2, bl2.reshape(1, -1))


@functools.cache
def _make_agg(N, E, F):
    CH = {128: 128, 256: 64, 512: 32}[F]
    SEG = 8000 if F <= 128 else 4000
    NP = 2 if F >= 512 else 1
    P = NC * NS * NP
    R = ((N + P - 1) // P + 7) // 8 * 8
    CAP = SEG + CH
    NSEG = (E + SEG - 1) // SEG
    assert E % SEG == 0
    mesh = plsc.VectorSubcoreMesh(core_axis_name="c", subcore_axis_name="s")

    @functools.partial(
        pl.kernel,
        out_type=jax.ShapeDtypeStruct((P * R, F), jnp.float32),
        mesh=mesh,
        compiler_params=pltpu.CompilerParams(needs_layout_passes=False),
        scratch_types=[
            pltpu.VMEM((SEG,), jnp.int32),
            pltpu.VMEM((SEG,), jnp.int32),
            pltpu.VMEM((CAP,), jnp.int32),
            pltpu.VMEM((CAP,), jnp.int32),
            pltpu.VMEM((CH,), jnp.int32),
            pltpu.VMEM((CH, F), jnp.float32),
            pltpu.VMEM((R + 8, F), jnp.float32),
            pltpu.SemaphoreType.DMA,
        ],
    )
    def agg(hs_hbm, src_hbm, dst_hbm, zrows_hbm, out_hbm,
            src_v, dst_v, csrc, cloc, gidx, gbuf, acc, gsem):
        c = lax.axis_index("c")
        s = lax.axis_index("s")
        for q in range(NP):
            pt = (c * NS + s) * NP + q
            lo = pt * R
            hi = jnp.minimum(lo + R, N)

            off = 0
            while off < R:
                n = min(128, R - off)
                pltpu.sync_copy(zrows_hbm.at[pl.ds(0, n)],
                                acc.at[pl.ds(off, n)])
                off += n

            def seg_body(g, _):
                pltpu.sync_copy(src_hbm.at[pl.ds(g * SEG, SEG)], src_v)
                pltpu.sync_copy(dst_hbm.at[pl.ds(g * SEG, SEG)], dst_v)

                def cbody(i, m):
                    s16 = src_v[pl.ds(i * L, L)]
                    d16 = dst_v[pl.ds(i * L, L)]
                    inb = (d16 >= lo) & (d16 < hi)
                    inc = plsc.cumsum(inb.astype(jnp.int32))
                    pos = m + inc - inb.astype(jnp.int32)
                    plsc.store_scatter(csrc, [pos], s16, mask=inb)
                    plsc.store_scatter(cloc, [pos], d16 - lo, mask=inb)
                    return m + inc[L - 1]

                m = lax.fori_loop(0, SEG // L, cbody, jnp.int32(0))

                pad16 = (c * NS + s) * L + lax.iota(jnp.int32, L)
                for t in range(CH // L):
                    csrc[pl.ds(m + t * L, L)] = pad16
                    cloc[pl.ds(m + t * L, L)] = jnp.full((L,), R, jnp.int32)

                nch = (m + CH - 1) // CH

                def chunk_body(j, _):
                    base = j * CH
                    for k in range(CH // L):
                        gidx[pl.ds(k * L, L)] = csrc[pl.ds(base + k * L, L)]
                    pltpu.async_copy(hs_hbm.at[gidx], gbuf, gsem).wait()
                    nv16 = (jnp.minimum(CH, m - base) + L - 1) // L

                    def row_body(i, _):
                        dl16 = cloc[pl.ds(base + i * L, L)]
                        for t in range(L):
                            dl = dl16[t]
                            for k in range(F // L):
                                sl = pl.ds(k * L, L)
                                acc[dl, sl] = acc[dl, sl] + gbuf[i * L + t, sl]
                        return 0

                    lax.fori_loop(0, nv16, row_body, 0)
                    return 0

                lax.fori_loop(0, nch, chunk_body, 0)
                return 0

            lax.fori_loop(0, NSEG, seg_body, 0)

            off = 0
            while off < R:
                n = min(256, R - off)
                pltpu.sync_copy(acc.at[pl.ds(off, n)],
                                out_hbm.at[pl.ds(lo + off, n)])
                off += n

    return agg, P * R


def _aggregate(hs, src, dst_masked):
    N, F = hs.shape
    E = src.shape[0]
    zrows = jnp.zeros((128, F), jnp.float32)
    agg, NPAD = _make_agg(N, E, F)
    outp = agg(hs, src, dst_masked, zrows)
    return lax.slice_in_dim(outp, 0, N)


@functools.cache
def _make_mis(N, E2):
    NPAD = 10240
    OWN = NPAD // NS
    EPT = E2 // NS
    BIG = N
    mesh = plsc.VectorSubcoreMesh(core_axis_name="c", subcore_axis_name="s",
                                  num_cores=1)
    i32 = jnp.int32

    @functools.partial(
        pl.kernel,
        out_type=jax.ShapeDtypeStruct((NPAD,), i32),
        mesh=mesh,
        compiler_params=pltpu.CompilerParams(needs_layout_passes=False),
        scratch_types=[
            pltpu.VMEM((EPT,), i32),
            pltpu.VMEM((NPAD,), i32),
            pltpu.VMEM((NPAD,), i32),
            pltpu.VMEM((NPAD,), i32),
            pltpu.VMEM((NPAD,), i32),
            pltpu.VMEM((OWN,), i32),
            pltpu.VMEM((OWN,), i32),
            pltpu.VMEM((OWN,), i32),
            pltpu.VMEM((NS, OWN), i32),
            pltpu.VMEM((NS, L), i32),
            pltpu.VMEM((L,), i32),
            pltpu.VMEM_SHARED((NS, NPAD), i32),
            pltpu.VMEM_SHARED((NPAD,), i32),
            pltpu.VMEM_SHARED((NS, L), i32),
        ],
    )
    def mis_k(packed_hbm, perm_hbm, v_hbm, mis_hbm,
              edges_v, rank_t, mask_t, killed_t, local_t,
              rank_own, local_own, mis_own, mbuf, fbuf, vbuf,
              pub, garr, gflags):
        t = lax.axis_index("s")
        own0 = t * OWN
        iota = lax.iota(i32, L)
        ones16 = jnp.ones((L,), i32)
        zeros16 = jnp.zeros((L,), i32)

        pltpu.sync_copy(packed_hbm.at[pl.ds(t * EPT, EPT)], edges_v)
        pltpu.sync_copy(v_hbm, vbuf)
        V = vbuf[pl.ds(0, L)][0]

        def ibody(j, _):
            idx16 = j * L + iota
            mask_t[pl.ds(j * L, L)] = (idx16 < V).astype(i32)
            return 0
        lax.fori_loop(0, NPAD // L, ibody, 0)

        def i2body(j, _):
            sl = pl.ds(j * L, L)
            rank_own[sl] = jnp.full((L,), BIG, i32)
            mis_own[sl] = zeros16
            return 0
        lax.fori_loop(0, OWN // L, i2body, 0)

        pltpu.sync_copy(perm_hbm, local_t.at[pl.ds(0, N)])

        def rbody(j, _):
            p16 = local_t[pl.ds(j * L, L)]
            inown = (p16 >= own0) & (p16 < own0 + OWN)
            plsc.store_scatter(rank_own, [p16 - own0], j * L + iota, mask=inown)
            return 0
        lax.fori_loop(0, N // L, rbody, 0)

        pltpu.sync_copy(rank_own, garr.at[pl.ds(own0, OWN)])
        plsc.subcore_barrier()
        pltpu.sync_copy(garr, rank_t)
        plsc.subcore_barrier()

        def zero_killed():
            def zbody(j, _):
                killed_t[pl.ds(j * L, L)] = zeros16
                return 0
            lax.fori_loop(0, NPAD // L, zbody, 0)

        def merge_or(dst_own):
            pltpu.sync_copy(pub.at[:, pl.ds(own0, OWN)], mbuf)

            def obody(j, _):
                sl = pl.ds(j * L, L)
                acc = zeros16
                for tt in range(NS):
                    acc = acc | mbuf[tt, sl]
                dst_own[sl] = acc
                return 0
            lax.fori_loop(0, OWN // L, obody, 0)

        def loop_body(go):
            zero_killed()

            def e1body(j, _):
                p16 = edges_v[pl.ds(j * L, L)]
                d16 = p16 & 16383
                s16 = lax.shift_right_logical(p16, 14)
                rs = plsc.load_gather(rank_t, [s16])
                rd = plsc.load_gather(rank_t, [d16])
                ms = plsc.load_gather(mask_t, [s16])
                ind = (ms > 0) & (rs < rd)
                plsc.store_scatter(killed_t, [d16], ones16, mask=ind)
                return 0
            lax.fori_loop(0, EPT // L, e1body, 0)

            pltpu.sync_copy(killed_t, pub.at[t])
            plsc.subcore_barrier()
            merge_or(local_own)

            def lbody(j, _):
                sl = pl.ds(j * L, L)
                loc = jnp.where(mask_t[pl.ds(own0 + j * L, L)] > 0,
                                1 - jnp.minimum(local_own[sl], 1), 0)
                local_own[sl] = loc
                mis_own[sl] = mis_own[sl] | loc
                return 0
            lax.fori_loop(0, OWN // L, lbody, 0)

            plsc.subcore_barrier()
            pltpu.sync_copy(local_own, garr.at[pl.ds(own0, OWN)])
            plsc.subcore_barrier()
            pltpu.sync_copy(garr, local_t)

            zero_killed()

            def e2body(j, _):
                p16 = edges_v[pl.ds(j * L, L)]
                d16 = p16 & 16383
                s16 = lax.shift_right_logical(p16, 14)
                ls = plsc.load_gather(local_t, [s16])
                plsc.store_scatter(killed_t, [d16], ones16, mask=ls > 0)
                return 0
            lax.fori_loop(0, EPT // L, e2body, 0)

            plsc.subcore_barrier()
            pltpu.sync_copy(killed_t, pub.at[t])
            plsc.subcore_barrier()
            merge_or(rank_own)

            anyv = zeros16

            def ubody(j, anyv):
                sl = pl.ds(j * L, L)
                newm = (mask_t[pl.ds(own0 + j * L, L)]
                        * (1 - local_own[sl])
                        * (1 - jnp.minimum(rank_own[sl], 1)))
                local_t[pl.ds(j * L, L)] = newm
                return anyv | newm
            anyv = lax.fori_loop(0, OWN // L, ubody, anyv)

            pltpu.sync_copy(local_t.at[pl.ds(0, OWN)], garr.at[pl.ds(own0, OWN)])
            fbuf[0, pl.ds(0, L)] = jnp.minimum(anyv, 1)
            pltpu.sync_copy(fbuf.at[0], gflags.at[t])
            plsc.subcore_barrier()
            pltpu.sync_copy(garr, mask_t)
            pltpu.sync_copy(gflags, fbuf)
            plsc.subcore_barrier()

            accv = zeros16
            for tt in range(NS):
                accv = accv | fbuf[tt, pl.ds(0, L)]
            return plsc.cummax(accv)[L - 1]

        lax.while_loop(lambda go: go > 0, loop_body, jnp.int32(1))

        pltpu.sync_copy(mis_own, mis_hbm.at[pl.ds(own0, OWN)])

    return mis_k


def _kmis(score, src, dst, N, V, node_valid, edge_valid):
    s = score.reshape(-1)
    s_eff = jnp.where(node_valid, s, -jnp.inf)
    perm = jnp.argsort(-s_eff).astype(jnp.int32)
    rank = jnp.zeros((N,), jnp.int32).at[perm].set(jnp.arange(N, dtype=jnp.int32))
    ss = jnp.concatenate([src, dst])
    dd = jnp.concatenate([dst, src])
    em = jnp.concatenate([edge_valid, edge_valid])
    BIG = jnp.int32(N)

    ssm = jnp.where(em, ss, N)
    ddm = jnp.where(em, dd, N)
    packed = ssm * jnp.int32(16384) + ddm
    misI = _make_mis(N, ss.shape[0])(
        packed, perm, jnp.full((16,), V, jnp.int32))
    mis = misI[:N] > 0
    r_mis = jnp.where(mis, rank, BIG)
    cand = jnp.full((N,), BIG, jnp.int32).at[dd].min(jnp.where(em, r_mis[ss], BIG))
    cand = jnp.minimum(cand, r_mis)
    cluster_node = perm[jnp.clip(cand, 0, N - 1)]
    Nc = jnp.sum(mis).astype(jnp.int32)
    new_id = jnp.where(mis, jnp.cumsum(mis.astype(jnp.int32)) - 1, 0)
    cluster = new_id[cluster_node]
    cu = cluster[src]
    cv = cluster[dst]
    keep = (cu != cv) & edge_valid
    SENT = jnp.int32(jnp.iinfo(jnp.int32).max)
    key = jnp.sort(jnp.where(keep, cu * Nc + cv, SENT))
    uniq = (key < SENT) & jnp.concatenate(
        [jnp.ones((1,), bool), key[1:] != key[:-1]])
    den = jnp.maximum(Nc, 1)
    new_src = jnp.where(uniq, key // den, 0).astype(jnp.int32)
    new_dst = jnp.where(uniq, key % den, 0).astype(jnp.int32)
    return mis, new_id, Nc, new_src, new_dst, uniq


def _dinv_of(deg):
    return jnp.where(deg > 0, lax.rsqrt(deg), 0.0)[:, None]


def kernel(x, edge_index, batch, W1, b1, ws1, bs1, W2, b2, ws2, bs2,
           W3, b3, Wl1, bl1, Wl2, bl2):
    src = edge_index[0]
    dst = edge_index[1]
    N = x.shape[0]
    E = src.shape[0]
    ones_n = jnp.ones((N,), bool)
    ones_e = jnp.ones((E,), bool)
    ones_col = jnp.ones((N, 1), jnp.float32)
    iota_n = jnp.arange(N, dtype=jnp.int32)

    deg1 = jnp.zeros((N,), jnp.float32).at[dst].add(1.0) + 1.0
    dinv1 = _dinv_of(deg1)
    hs1 = _mm_scale(x, W1, dinv1)
    acc1 = _aggregate(hs1, src, dst)
    h, s1 = _epilogue(acc1, hs1, dinv1, ones_col, b1, ws1)
    s1 = s1 + bs1

    mis1, nid1, Nc1, src1, dst1, ev2 = _kmis(s1, src, dst, N, jnp.int32(N),
                                             ones_n, ones_e)
    idx1 = jnp.where(mis1, nid1, N)
    val1 = h * s1
    h = jnp.zeros_like(val1).at[idx1].set(val1, mode="drop")
    nv2 = iota_n < Nc1
    nv2f = nv2.astype(jnp.float32)

    dstm2 = jnp.where(ev2, dst1, N)
    deg2 = (jnp.zeros((N,), jnp.float32)
            .at[dst1].add(jnp.where(ev2, 1.0, 0.0)) + nv2f)
    dinv2 = _dinv_of(deg2)
    hs2 = _mm_scale(h, W2, dinv2)
    acc2 = _aggregate(hs2, src1, dstm2)
    h, s2 = _epilogue(acc2, hs2, dinv2, nv2f[:, None], b2, ws2)
    s2 = s2 + bs2

    mis2, nid2, Nc2, src2, dst2, ev3 = _kmis(s2, src1, dst1, N, Nc1, nv2, ev2)
    idx2 = jnp.where(mis2, nid2, N)
    val2 = h * s2
    h = jnp.zeros_like(val2).at[idx2].set(val2, mode="drop")
    nv3 = iota_n < Nc2
    nv3f = nv3.astype(jnp.float32)

    dstm3 = jnp.where(ev3, dst2, N)
    deg3 = (jnp.zeros((N,), jnp.float32)
            .at[dst2].add(jnp.where(ev3, 1.0, 0.0)) + nv3f)
    dinv3 = _dinv_of(deg3)
    hs3 = _mm_scale(h, W3, dinv3)
    acc3 = _aggregate(hs3, src2, dstm3)
    h, _ = _epilogue(acc3, hs3, dinv3, nv3f[:, None], b3,
                     jnp.zeros((W3.shape[1], 1), jnp.float32))

    return _head(h, nv3f, Wl1, bl1, Wl2, bl2)


# speedup vs baseline: 7.8100x; 1.3478x over previous
import functools

import jax
import jax.numpy as jnp
from jax import lax
from jax.experimental import pallas as pl
from jax.experimental.pallas import tpu as pltpu
from jax.experimental.pallas import tpu_sc as plsc

NC = 2
NS = 16
L = 16


def _mm_scale_body(x_ref, w_ref, dinv_ref, o_ref):
    hw = jnp.dot(x_ref[...], w_ref[...], preferred_element_type=jnp.float32)
    o_ref[...] = dinv_ref[...] * hw


def _mm_scale(x, W, dinv, block_m=2000):
    M, K = x.shape
    _, N = W.shape
    return pl.pallas_call(
        _mm_scale_body,
        grid=(M // block_m,),
        in_specs=[
            pl.BlockSpec((block_m, K), lambda i: (i, 0)),
            pl.BlockSpec((K, N), lambda i: (0, 0)),
            pl.BlockSpec((block_m, 1), lambda i: (i, 0)),
        ],
        out_specs=pl.BlockSpec((block_m, N), lambda i: (i, 0)),
        out_shape=jax.ShapeDtypeStruct((M, N), jnp.float32),
    )(x, W, dinv)


def _epi_body(acc_ref, hs_ref, dinv_ref, nv_ref, b_ref, ws_ref, h_ref, s_ref):
    h = dinv_ref[...] * (acc_ref[...] + hs_ref[...]) + b_ref[...]
    h = jnp.maximum(h, 0.0) * nv_ref[...]
    h_ref[...] = h
    s_ref[...] = jnp.dot(h, ws_ref[...], preferred_element_type=jnp.float32)


def _epilogue(acc, hs, dinv, nv, b, ws, block_m=2000):
    M, N = acc.shape
    return pl.pallas_call(
        _epi_body,
        grid=(M // block_m,),
        in_specs=[
            pl.BlockSpec((block_m, N), lambda i: (i, 0)),
            pl.BlockSpec((block_m, N), lambda i: (i, 0)),
            pl.BlockSpec((block_m, 1), lambda i: (i, 0)),
            pl.BlockSpec((block_m, 1), lambda i: (i, 0)),
            pl.BlockSpec((1, N), lambda i: (0, 0)),
            pl.BlockSpec((N, 1), lambda i: (0, 0)),
        ],
        out_specs=(pl.BlockSpec((block_m, N), lambda i: (i, 0)),
                   pl.BlockSpec((block_m, 1), lambda i: (i, 0))),
        out_shape=(jax.ShapeDtypeStruct((M, N), jnp.float32),
                   jax.ShapeDtypeStruct((M, 1), jnp.float32)),
    )(acc, hs, dinv, nv, b.reshape(1, -1), ws)


def _head_body(h_ref, nv_ref, wl1_ref, bl1_ref, wl2_ref, bl2_ref, o_ref):
    h = h_ref[...]
    nv = nv_ref[...]
    gmax = jnp.max(jnp.where(nv > 0, h, -jnp.inf), axis=0, keepdims=True)
    gsum = jnp.sum(h, axis=0, keepdims=True)
    cnt = jnp.maximum(jnp.sum(nv), 1.0)
    g = jnp.concatenate([gmax, gsum / cnt], axis=1)
    z = jnp.maximum(jnp.dot(g, wl1_ref[...], preferred_element_type=jnp.float32)
                    + bl1_ref[...], 0.0)
    logits = jnp.dot(z, wl2_ref[...], preferred_element_type=jnp.float32) + bl2_ref[...]
    o_ref[...] = jax.nn.log_softmax(logits, axis=-1)


def _head(h, nv_f32, Wl1, bl1, Wl2, bl2):
    M, _ = h.shape
    return pl.pallas_call(
        _head_body,
        out_shape=jax.ShapeDtypeStruct((1, Wl2.shape[1]), jnp.float32),
    )(h, nv_f32.reshape(M, 1), Wl1, bl1.reshape(1, -1), Wl2, bl2.reshape(1, -1))


@functools.cache
def _make_agg(N, E, F):
    CH = {128: 128, 256: 64, 512: 32}[F]
    SEG = 8000 if F <= 128 else 4000
    NP = 2 if F >= 512 else 1
    P = NC * NS * NP
    R = ((N + P - 1) // P + 7) // 8 * 8
    CAP = SEG + CH
    NSEG = (E + SEG - 1) // SEG
    assert E % SEG == 0
    mesh = plsc.VectorSubcoreMesh(core_axis_name="c", subcore_axis_name="s")

    @functools.partial(
        pl.kernel,
        out_type=jax.ShapeDtypeStruct((P * R, F), jnp.float32),
        mesh=mesh,
        compiler_params=pltpu.CompilerParams(needs_layout_passes=False),
        scratch_types=[
            pltpu.VMEM((SEG,), jnp.int32),
            pltpu.VMEM((SEG,), jnp.int32),
            pltpu.VMEM((CAP,), jnp.int32),
            pltpu.VMEM((CAP,), jnp.int32),
            pltpu.VMEM((CH,), jnp.int32),
            pltpu.VMEM((CH, F), jnp.float32),
            pltpu.VMEM((R + 8, F), jnp.float32),
            pltpu.SemaphoreType.DMA,
        ],
    )
    def agg(hs_hbm, src_hbm, dst_hbm, zrows_hbm, out_hbm,
            src_v, dst_v, csrc, cloc, gidx, gbuf, acc, gsem):
        c = lax.axis_index("c")
        s = lax.axis_index("s")
        for q in range(NP):
            pt = (c * NS + s) * NP + q
            shift = {32: 5, 64: 6}[P]

            off = 0
            while off < R:
                n = min(128, R - off)
                pltpu.sync_copy(zrows_hbm.at[pl.ds(0, n)],
                                acc.at[pl.ds(off, n)])
                off += n

            def seg_body(g, _):
                pltpu.sync_copy(src_hbm.at[pl.ds(g * SEG, SEG)], src_v)
                pltpu.sync_copy(dst_hbm.at[pl.ds(g * SEG, SEG)], dst_v)

                def cbody(i, m):
                    s16 = src_v[pl.ds(i * L, L)]
                    d16 = dst_v[pl.ds(i * L, L)]
                    g16 = lax.shift_right_logical(d16, 3)
                    inb = ((g16 & (P - 1)) == pt) & (d16 < N)
                    dloc = (lax.shift_left(
                        lax.shift_right_logical(g16, shift), 3)
                        | (d16 & 7))
                    inc = plsc.cumsum(inb.astype(jnp.int32))
                    pos = m + inc - inb.astype(jnp.int32)
                    plsc.store_scatter(csrc, [pos], s16, mask=inb)
                    plsc.store_scatter(cloc, [pos], dloc, mask=inb)
                    return m + inc[L - 1]

                m = lax.fori_loop(0, SEG // L, cbody, jnp.int32(0))

                pad16 = (c * NS + s) * L + lax.iota(jnp.int32, L)
                for t in range(CH // L):
                    csrc[pl.ds(m + t * L, L)] = pad16
                    cloc[pl.ds(m + t * L, L)] = jnp.full((L,), R, jnp.int32)

                nch = (m + CH - 1) // CH

                def chunk_body(j, _):
                    base = j * CH
                    for k in range(CH // L):
                        gidx[pl.ds(k * L, L)] = csrc[pl.ds(base + k * L, L)]
                    pltpu.async_copy(hs_hbm.at[gidx], gbuf, gsem).wait()
                    nv16 = (jnp.minimum(CH, m - base) + L - 1) // L

                    def row_body(i, _):
                        dl16 = cloc[pl.ds(base + i * L, L)]
                        for t in range(L):
                            dl = dl16[t]
                            for k in range(F // L):
                                sl = pl.ds(k * L, L)
                                acc[dl, sl] = acc[dl, sl] + gbuf[i * L + t, sl]
                        return 0

                    lax.fori_loop(0, nv16, row_body, 0)
                    return 0

                lax.fori_loop(0, nch, chunk_body, 0)
                return 0

            lax.fori_loop(0, NSEG, seg_body, 0)

            off = 0
            while off < R:
                n = min(256, R - off)
                pltpu.sync_copy(acc.at[pl.ds(off, n)],
                                out_hbm.at[pl.ds(pt * R + off, n)])
                off += n

    return agg, P, R


def _aggregate(hs, src, dst_masked):
    N, F = hs.shape
    E = src.shape[0]
    zrows = jnp.zeros((128, F), jnp.float32)
    agg, P, R = _make_agg(N, E, F)
    outp = agg(hs, src, dst_masked, zrows)
    outp = outp.reshape(P, R // 8, 8, F).transpose(1, 0, 2, 3).reshape(-1, F)
    return lax.slice_in_dim(outp, 0, N)


@functools.cache
def _make_mis(N, E2):
    NPAD = 10240
    OWN = NPAD // NS
    EPT = E2 // NS
    BIG = N
    mesh = plsc.VectorSubcoreMesh(core_axis_name="c", subcore_axis_name="s",
                                  num_cores=1)
    i32 = jnp.int32

    @functools.partial(
        pl.kernel,
        out_type=jax.ShapeDtypeStruct((NPAD,), i32),
        mesh=mesh,
        compiler_params=pltpu.CompilerParams(needs_layout_passes=False),
        scratch_types=[
            pltpu.VMEM((EPT,), i32),
            pltpu.VMEM((NPAD,), i32),
            pltpu.VMEM((NPAD,), i32),
            pltpu.VMEM((NPAD,), i32),
            pltpu.VMEM((NPAD,), i32),
            pltpu.VMEM((OWN,), i32),
            pltpu.VMEM((OWN,), i32),
            pltpu.VMEM((OWN,), i32),
            pltpu.VMEM((NS, OWN), i32),
            pltpu.VMEM((NS, L), i32),
            pltpu.VMEM((L,), i32),
            pltpu.VMEM_SHARED((NS, NPAD), i32),
            pltpu.VMEM_SHARED((NPAD,), i32),
            pltpu.VMEM_SHARED((NS, L), i32),
        ],
    )
    def mis_k(packed_hbm, perm_hbm, v_hbm, mis_hbm,
              edges_v, rank_t, mask_t, killed_t, local_t,
              rank_own, local_own, mis_own, mbuf, fbuf, vbuf,
              pub, garr, gflags):
        t = lax.axis_index("s")
        own0 = t * OWN
        iota = lax.iota(i32, L)
        ones16 = jnp.ones((L,), i32)
        zeros16 = jnp.zeros((L,), i32)

        pltpu.sync_copy(packed_hbm.at[pl.ds(t * EPT, EPT)], edges_v)
        pltpu.sync_copy(v_hbm, vbuf)
        V = vbuf[pl.ds(0, L)][0]

        def ibody(j, _):
            idx16 = j * L + iota
            mask_t[pl.ds(j * L, L)] = (idx16 < V).astype(i32)
            return 0
        lax.fori_loop(0, NPAD // L, ibody, 0)

        def i2body(j, _):
            sl = pl.ds(j * L, L)
            rank_own[sl] = jnp.full((L,), BIG, i32)
            mis_own[sl] = zeros16
            return 0
        lax.fori_loop(0, OWN // L, i2body, 0)

        pltpu.sync_copy(perm_hbm, local_t.at[pl.ds(0, N)])

        def rbody(j, _):
            p16 = local_t[pl.ds(j * L, L)]
            inown = (p16 >= own0) & (p16 < own0 + OWN)
            plsc.store_scatter(rank_own, [p16 - own0], j * L + iota, mask=inown)
            return 0
        lax.fori_loop(0, N // L, rbody, 0)

        pltpu.sync_copy(rank_own, garr.at[pl.ds(own0, OWN)])
        plsc.subcore_barrier()
        pltpu.sync_copy(garr, rank_t)
        plsc.subcore_barrier()

        def zero_killed():
            def zbody(j, _):
                killed_t[pl.ds(j * L, L)] = zeros16
                return 0
            lax.fori_loop(0, NPAD // L, zbody, 0)

        def merge_or(dst_own):
            pltpu.sync_copy(pub.at[:, pl.ds(own0, OWN)], mbuf)

            def obody(j, _):
                sl = pl.ds(j * L, L)
                acc = zeros16
                for tt in range(NS):
                    acc = acc | mbuf[tt, sl]
                dst_own[sl] = acc
                return 0
            lax.fori_loop(0, OWN // L, obody, 0)

        def loop_body(go):
            zero_killed()

            def e1body(j, _):
                p16 = edges_v[pl.ds(j * L, L)]
                d16 = p16 & 16383
                s16 = lax.shift_right_logical(p16, 14)
                rs = plsc.load_gather(rank_t, [s16])
                rd = plsc.load_gather(rank_t, [d16])
                ms = plsc.load_gather(mask_t, [s16])
                ind = (ms > 0) & (rs < rd)
                plsc.store_scatter(killed_t, [d16], ones16, mask=ind)
                return 0
            lax.fori_loop(0, EPT // L, e1body, 0)

            pltpu.sync_copy(killed_t, pub.at[t])
            plsc.subcore_barrier()
            merge_or(local_own)

            def lbody(j, _):
                sl = pl.ds(j * L, L)
                loc = jnp.where(mask_t[pl.ds(own0 + j * L, L)] > 0,
                                1 - jnp.minimum(local_own[sl], 1), 0)
                local_own[sl] = loc
                mis_own[sl] = mis_own[sl] | loc
                return 0
            lax.fori_loop(0, OWN // L, lbody, 0)

            plsc.subcore_barrier()
            pltpu.sync_copy(local_own, garr.at[pl.ds(own0, OWN)])
            plsc.subcore_barrier()
            pltpu.sync_copy(garr, local_t)

            zero_killed()

            def e2body(j, _):
                p16 = edges_v[pl.ds(j * L, L)]
                d16 = p16 & 16383
                s16 = lax.shift_right_logical(p16, 14)
                ls = plsc.load_gather(local_t, [s16])
                plsc.store_scatter(killed_t, [d16], ones16, mask=ls > 0)
                return 0
            lax.fori_loop(0, EPT // L, e2body, 0)

            plsc.subcore_barrier()
            pltpu.sync_copy(killed_t, pub.at[t])
            plsc.subcore_barrier()
            merge_or(rank_own)

            anyv = zeros16

            def ubody(j, anyv):
                sl = pl.ds(j * L, L)
                newm = (mask_t[pl.ds(own0 + j * L, L)]
                        * (1 - local_own[sl])
                        * (1 - jnp.minimum(rank_own[sl], 1)))
                local_t[pl.ds(j * L, L)] = newm
                return anyv | newm
            anyv = lax.fori_loop(0, OWN // L, ubody, anyv)

            pltpu.sync_copy(local_t.at[pl.ds(0, OWN)], garr.at[pl.ds(own0, OWN)])
            fbuf[0, pl.ds(0, L)] = jnp.minimum(anyv, 1)
            pltpu.sync_copy(fbuf.at[0], gflags.at[t])
            plsc.subcore_barrier()
            pltpu.sync_copy(garr, mask_t)
            pltpu.sync_copy(gflags, fbuf)
            plsc.subcore_barrier()

            accv = zeros16
            for tt in range(NS):
                accv = accv | fbuf[tt, pl.ds(0, L)]
            return plsc.cummax(accv)[L - 1]

        lax.while_loop(lambda go: go > 0, loop_body, jnp.int32(1))

        pltpu.sync_copy(mis_own, mis_hbm.at[pl.ds(own0, OWN)])

    return mis_k


def _kmis(score, src, dst, N, V, node_valid, edge_valid):
    s = score.reshape(-1)
    s_eff = jnp.where(node_valid, s, -jnp.inf)
    perm = jnp.argsort(-s_eff).astype(jnp.int32)
    rank = jnp.zeros((N,), jnp.int32).at[perm].set(jnp.arange(N, dtype=jnp.int32))
    ss = jnp.concatenate([src, dst])
    dd = jnp.concatenate([dst, src])
    em = jnp.concatenate([edge_valid, edge_valid])
    BIG = jnp.int32(N)

    ssm = jnp.where(em, ss, N)
    ddm = jnp.where(em, dd, N)
    packed = ssm * jnp.int32(16384) + ddm
    misI = _make_mis(N, ss.shape[0])(
        packed, perm, jnp.full((16,), V, jnp.int32))
    mis = misI[:N] > 0
    r_mis = jnp.where(mis, rank, BIG)
    cand = jnp.full((N,), BIG, jnp.int32).at[dd].min(jnp.where(em, r_mis[ss], BIG))
    cand = jnp.minimum(cand, r_mis)
    cluster_node = perm[jnp.clip(cand, 0, N - 1)]
    Nc = jnp.sum(mis).astype(jnp.int32)
    new_id = jnp.where(mis, jnp.cumsum(mis.astype(jnp.int32)) - 1, 0)
    cluster = new_id[cluster_node]
    cu = cluster[src]
    cv = cluster[dst]
    keep = (cu != cv) & edge_valid
    SENT = jnp.int32(jnp.iinfo(jnp.int32).max)
    key = jnp.sort(jnp.where(keep, cu * Nc + cv, SENT))
    uniq = (key < SENT) & jnp.concatenate(
        [jnp.ones((1,), bool), key[1:] != key[:-1]])
    den = jnp.maximum(Nc, 1)
    new_src = jnp.where(uniq, key // den, 0).astype(jnp.int32)
    new_dst = jnp.where(uniq, key % den, 0).astype(jnp.int32)
    return mis, new_id, Nc, new_src, new_dst, uniq


def _dinv_of(deg):
    return jnp.where(deg > 0, lax.rsqrt(deg), 0.0)[:, None]


def kernel(x, edge_index, batch, W1, b1, ws1, bs1, W2, b2, ws2, bs2,
           W3, b3, Wl1, bl1, Wl2, bl2):
    src = edge_index[0]
    dst = edge_index[1]
    N = x.shape[0]
    E = src.shape[0]
    ones_n = jnp.ones((N,), bool)
    ones_e = jnp.ones((E,), bool)
    ones_col = jnp.ones((N, 1), jnp.float32)
    iota_n = jnp.arange(N, dtype=jnp.int32)

    deg1 = jnp.zeros((N,), jnp.float32).at[dst].add(1.0) + 1.0
    dinv1 = _dinv_of(deg1)
    hs1 = _mm_scale(x, W1, dinv1)
    acc1 = _aggregate(hs1, src, dst)
    h, s1 = _epilogue(acc1, hs1, dinv1, ones_col, b1, ws1)
    s1 = s1 + bs1

    mis1, nid1, Nc1, src1, dst1, ev2 = _kmis(s1, src, dst, N, jnp.int32(N),
                                             ones_n, ones_e)
    idx1 = jnp.where(mis1, nid1, N)
    val1 = h * s1
    h = jnp.zeros_like(val1).at[idx1].set(val1, mode="drop")
    nv2 = iota_n < Nc1
    nv2f = nv2.astype(jnp.float32)

    dstm2 = jnp.where(ev2, dst1, N)
    deg2 = (jnp.zeros((N,), jnp.float32)
            .at[dst1].add(jnp.where(ev2, 1.0, 0.0)) + nv2f)
    dinv2 = _dinv_of(deg2)
    hs2 = _mm_scale(h, W2, dinv2)
    acc2 = _aggregate(hs2, src1, dstm2)
    h, s2 = _epilogue(acc2, hs2, dinv2, nv2f[:, None], b2, ws2)
    s2 = s2 + bs2

    mis2, nid2, Nc2, src2, dst2, ev3 = _kmis(s2, src1, dst1, N, Nc1, nv2, ev2)
    idx2 = jnp.where(mis2, nid2, N)
    val2 = h * s2
    h = jnp.zeros_like(val2).at[idx2].set(val2, mode="drop")
    nv3 = iota_n < Nc2
    nv3f = nv3.astype(jnp.float32)

    dstm3 = jnp.where(ev3, dst2, N)
    deg3 = (jnp.zeros((N,), jnp.float32)
            .at[dst2].add(jnp.where(ev3, 1.0, 0.0)) + nv3f)
    dinv3 = _dinv_of(deg3)
    hs3 = _mm_scale(h, W3, dinv3)
    acc3 = _aggregate(hs3, src2, dstm3)
    h, _ = _epilogue(acc3, hs3, dinv3, nv3f[:, None], b3,
                     jnp.zeros((W3.shape[1], 1), jnp.float32))

    return _head(h, nv3f, Wl1, bl1, Wl2, bl2)


# speedup vs baseline: 15.1534x; 1.9402x over previous
import functools

import jax
import jax.numpy as jnp
from jax import lax
from jax.experimental import pallas as pl
from jax.experimental.pallas import tpu as pltpu
from jax.experimental.pallas import tpu_sc as plsc

NC = 2
NS = 16
L = 16


def _mm_scale_body(x_ref, w_ref, dinv_ref, o_ref):
    hw = jnp.dot(x_ref[...], w_ref[...], preferred_element_type=jnp.float32)
    o_ref[...] = dinv_ref[...] * hw


def _mm_scale(x, W, dinv, block_m=2000):
    M, K = x.shape
    _, N = W.shape
    return pl.pallas_call(
        _mm_scale_body,
        grid=(M // block_m,),
        in_specs=[
            pl.BlockSpec((block_m, K), lambda i: (i, 0)),
            pl.BlockSpec((K, N), lambda i: (0, 0)),
            pl.BlockSpec((block_m, 1), lambda i: (i, 0)),
        ],
        out_specs=pl.BlockSpec((block_m, N), lambda i: (i, 0)),
        out_shape=jax.ShapeDtypeStruct((M, N), jnp.float32),
    )(x, W, dinv)


def _epi_body(acc_ref, hs_ref, dinv_ref, nv_ref, b_ref, ws_ref, h_ref, s_ref):
    h = dinv_ref[...] * (acc_ref[...] + hs_ref[...]) + b_ref[...]
    h = jnp.maximum(h, 0.0) * nv_ref[...]
    h_ref[...] = h
    s_ref[...] = jnp.dot(h, ws_ref[...], preferred_element_type=jnp.float32)


def _epilogue(acc, hs, dinv, nv, b, ws, block_m=2000):
    M, N = acc.shape
    return pl.pallas_call(
        _epi_body,
        grid=(M // block_m,),
        in_specs=[
            pl.BlockSpec((block_m, N), lambda i: (i, 0)),
            pl.BlockSpec((block_m, N), lambda i: (i, 0)),
            pl.BlockSpec((block_m, 1), lambda i: (i, 0)),
            pl.BlockSpec((block_m, 1), lambda i: (i, 0)),
            pl.BlockSpec((1, N), lambda i: (0, 0)),
            pl.BlockSpec((N, 1), lambda i: (0, 0)),
        ],
        out_specs=(pl.BlockSpec((block_m, N), lambda i: (i, 0)),
                   pl.BlockSpec((block_m, 1), lambda i: (i, 0))),
        out_shape=(jax.ShapeDtypeStruct((M, N), jnp.float32),
                   jax.ShapeDtypeStruct((M, 1), jnp.float32)),
    )(acc, hs, dinv, nv, b.reshape(1, -1), ws)


def _head_body(h_ref, nv_ref, wl1_ref, bl1_ref, wl2_ref, bl2_ref, o_ref):
    h = h_ref[...]
    nv = nv_ref[...]
    gmax = jnp.max(jnp.where(nv > 0, h, -jnp.inf), axis=0, keepdims=True)
    gsum = jnp.sum(h, axis=0, keepdims=True)
    cnt = jnp.maximum(jnp.sum(nv), 1.0)
    g = jnp.concatenate([gmax, gsum / cnt], axis=1)
    z = jnp.maximum(jnp.dot(g, wl1_ref[...], preferred_element_type=jnp.float32)
                    + bl1_ref[...], 0.0)
    logits = jnp.dot(z, wl2_ref[...], preferred_element_type=jnp.float32) + bl2_ref[...]
    o_ref[...] = jax.nn.log_softmax(logits, axis=-1)


def _head(h, nv_f32, Wl1, bl1, Wl2, bl2):
    M, _ = h.shape
    return pl.pallas_call(
        _head_body,
        out_shape=jax.ShapeDtypeStruct((1, Wl2.shape[1]), jnp.float32),
    )(h, nv_f32.reshape(M, 1), Wl1, bl1.reshape(1, -1), Wl2, bl2.reshape(1, -1))


@functools.cache
def _make_agg(N, E, F):
    CH = {128: 128, 256: 64, 512: 32}[F]
    SEG = 8000 if F <= 128 else 4000
    NP = 2 if F >= 512 else 1
    P = NC * NS * NP
    R = ((N + P - 1) // P + 7) // 8 * 8
    CAP = SEG + CH
    NSEG = (E + SEG - 1) // SEG
    assert E % SEG == 0
    mesh = plsc.VectorSubcoreMesh(core_axis_name="c", subcore_axis_name="s")

    @functools.partial(
        pl.kernel,
        out_type=jax.ShapeDtypeStruct((P * R, F), jnp.float32),
        mesh=mesh,
        compiler_params=pltpu.CompilerParams(needs_layout_passes=False),
        scratch_types=[
            pltpu.VMEM((SEG,), jnp.int32),
            pltpu.VMEM((SEG,), jnp.int32),
            pltpu.VMEM((CAP,), jnp.int32),
            pltpu.VMEM((CAP,), jnp.int32),
            pltpu.VMEM((CH,), jnp.int32),
            pltpu.VMEM((CH, F), jnp.float32),
            pltpu.VMEM((R + 8, F), jnp.float32),
            pltpu.SemaphoreType.DMA,
        ],
    )
    def agg(hs_hbm, src_hbm, dst_hbm, zrows_hbm, out_hbm,
            src_v, dst_v, csrc, cloc, gidx, gbuf, acc, gsem):
        c = lax.axis_index("c")
        s = lax.axis_index("s")
        for q in range(NP):
            pt = (c * NS + s) * NP + q
            shift = {32: 5, 64: 6}[P]

            off = 0
            while off < R:
                n = min(128, R - off)
                pltpu.sync_copy(zrows_hbm.at[pl.ds(0, n)],
                                acc.at[pl.ds(off, n)])
                off += n

            def seg_body(g, _):
                pltpu.sync_copy(src_hbm.at[pl.ds(g * SEG, SEG)], src_v)
                pltpu.sync_copy(dst_hbm.at[pl.ds(g * SEG, SEG)], dst_v)

                def cbody(i, m):
                    s16 = src_v[pl.ds(i * L, L)]
                    d16 = dst_v[pl.ds(i * L, L)]
                    g16 = lax.shift_right_logical(d16, 3)
                    inb = ((g16 & (P - 1)) == pt) & (d16 < N)
                    dloc = (lax.shift_left(
                        lax.shift_right_logical(g16, shift), 3)
                        | (d16 & 7))
                    inc = plsc.cumsum(inb.astype(jnp.int32))
                    pos = m + inc - inb.astype(jnp.int32)
                    plsc.store_scatter(csrc, [pos], s16, mask=inb)
                    plsc.store_scatter(cloc, [pos], dloc, mask=inb)
                    return m + inc[L - 1]

                m = lax.fori_loop(0, SEG // L, cbody, jnp.int32(0))

                pad16 = (c * NS + s) * L + lax.iota(jnp.int32, L)
                for t in range(CH // L):
                    csrc[pl.ds(m + t * L, L)] = pad16
                    cloc[pl.ds(m + t * L, L)] = jnp.full((L,), R, jnp.int32)

                nch = (m + CH - 1) // CH

                def chunk_body(j, _):
                    base = j * CH
                    for k in range(CH // L):
                        gidx[pl.ds(k * L, L)] = csrc[pl.ds(base + k * L, L)]
                    pltpu.async_copy(hs_hbm.at[gidx], gbuf, gsem).wait()
                    nv16 = (jnp.minimum(CH, m - base) + L - 1) // L

                    def row_body(i, _):
                        dl16 = cloc[pl.ds(base + i * L, L)]
                        for t in range(L):
                            dl = dl16[t]
                            for k in range(F // L):
                                sl = pl.ds(k * L, L)
                                acc[dl, sl] = acc[dl, sl] + gbuf[i * L + t, sl]
                        return 0

                    lax.fori_loop(0, nv16, row_body, 0)
                    return 0

                lax.fori_loop(0, nch, chunk_body, 0)
                return 0

            lax.fori_loop(0, NSEG, seg_body, 0)

            off = 0
            while off < R:
                n = min(256, R - off)
                pltpu.sync_copy(acc.at[pl.ds(off, n)],
                                out_hbm.at[pl.ds(pt * R + off, n)])
                off += n

    return agg, P, R


def _aggregate(hs, src, dst_masked):
    N, F = hs.shape
    E = src.shape[0]
    zrows = jnp.zeros((128, F), jnp.float32)
    agg, P, R = _make_agg(N, E, F)
    outp = agg(hs, src, dst_masked, zrows)
    outp = outp.reshape(P, R // 8, 8, F).transpose(1, 0, 2, 3).reshape(-1, F)
    return lax.slice_in_dim(outp, 0, N)


@functools.cache
def _make_mis(N, E2):
    NPAD = 10240
    OWN = NPAD // NS
    EPT = E2 // NS
    BIG = N
    mesh = plsc.VectorSubcoreMesh(core_axis_name="c", subcore_axis_name="s",
                                  num_cores=1)
    i32 = jnp.int32

    @functools.partial(
        pl.kernel,
        out_type=(jax.ShapeDtypeStruct((NPAD,), i32),
                  jax.ShapeDtypeStruct((NPAD,), i32),
                  jax.ShapeDtypeStruct((NPAD,), i32)),
        mesh=mesh,
        compiler_params=pltpu.CompilerParams(needs_layout_passes=False),
        scratch_types=[
            pltpu.VMEM((EPT,), i32),
            pltpu.VMEM((NPAD,), i32),
            pltpu.VMEM((NPAD,), i32),
            pltpu.VMEM((NPAD,), i32),
            pltpu.VMEM((NPAD,), i32),
            pltpu.VMEM((OWN,), i32),
            pltpu.VMEM((OWN,), i32),
            pltpu.VMEM((OWN,), i32),
            pltpu.VMEM((NS, OWN), i32),
            pltpu.VMEM((NS, L), i32),
            pltpu.VMEM((L,), i32),
            pltpu.VMEM((48,), i32),
            pltpu.VMEM((48,), i32),
            pltpu.VMEM_SHARED((NS, NPAD), i32),
            pltpu.VMEM_SHARED((NPAD,), i32),
            pltpu.VMEM_SHARED((NS, L), i32),
        ],
    )
    def mis_k(packed_hbm, perm_hbm, v_hbm, mis_hbm, rank_hbm, cand_hbm,
              edges_v, rank_t, mask_t, killed_t, local_t,
              rank_own, local_own, mis_own, mbuf, fbuf, vbuf, kbuf, vsbuf,
              pub, garr, gflags):
        t = lax.axis_index("s")
        own0 = t * OWN
        iota = lax.iota(i32, L)
        ones16 = jnp.ones((L,), i32)
        zeros16 = jnp.zeros((L,), i32)

        pltpu.sync_copy(packed_hbm.at[pl.ds(t * EPT, EPT)], edges_v)
        pltpu.sync_copy(v_hbm, vbuf)
        V = vbuf[pl.ds(0, L)][0]

        def ibody(j, _):
            idx16 = j * L + iota
            mask_t[pl.ds(j * L, L)] = (idx16 < V).astype(i32)
            return 0
        lax.fori_loop(0, NPAD // L, ibody, 0)

        def i2body(j, _):
            sl = pl.ds(j * L, L)
            rank_own[sl] = jnp.full((L,), BIG, i32)
            mis_own[sl] = zeros16
            return 0
        lax.fori_loop(0, OWN // L, i2body, 0)

        pltpu.sync_copy(perm_hbm, local_t.at[pl.ds(0, N)])

        def rbody(j, _):
            p16 = local_t[pl.ds(j * L, L)]
            inown = (p16 >= own0) & (p16 < own0 + OWN)
            plsc.store_scatter(rank_own, [p16 - own0], j * L + iota, mask=inown)
            return 0
        lax.fori_loop(0, N // L, rbody, 0)

        pltpu.sync_copy(rank_own, garr.at[pl.ds(own0, OWN)])
        plsc.subcore_barrier()
        pltpu.sync_copy(garr, rank_t)
        pltpu.sync_copy(rank_t.at[pl.ds(own0, OWN)],
                        rank_hbm.at[pl.ds(own0, OWN)])
        plsc.subcore_barrier()

        def zero_killed():
            def zbody(j, _):
                killed_t[pl.ds(j * L, L)] = zeros16
                return 0
            lax.fori_loop(0, NPAD // L, zbody, 0)

        def merge_or(dst_own):
            pltpu.sync_copy(pub.at[:, pl.ds(own0, OWN)], mbuf)

            def obody(j, _):
                sl = pl.ds(j * L, L)
                acc = zeros16
                for tt in range(NS):
                    acc = acc | mbuf[tt, sl]
                dst_own[sl] = acc
                return 0
            lax.fori_loop(0, OWN // L, obody, 0)

        def loop_body(go):
            zero_killed()

            def e1body(j, _):
                p16 = edges_v[pl.ds(j * L, L)]
                d16 = p16 & 16383
                s16 = lax.shift_right_logical(p16, 14)
                rs = plsc.load_gather(rank_t, [s16])
                rd = plsc.load_gather(rank_t, [d16])
                ms = plsc.load_gather(mask_t, [s16])
                ind = (ms > 0) & (rs < rd)
                plsc.store_scatter(killed_t, [d16], ones16, mask=ind)
                return 0
            lax.fori_loop(0, EPT // L, e1body, 0)

            pltpu.sync_copy(killed_t, pub.at[t])
            plsc.subcore_barrier()
            merge_or(local_own)

            def lbody(j, _):
                sl = pl.ds(j * L, L)
                loc = jnp.where(mask_t[pl.ds(own0 + j * L, L)] > 0,
                                1 - jnp.minimum(local_own[sl], 1), 0)
                local_own[sl] = loc
                mis_own[sl] = mis_own[sl] | loc
                return 0
            lax.fori_loop(0, OWN // L, lbody, 0)

            plsc.subcore_barrier()
            pltpu.sync_copy(local_own, garr.at[pl.ds(own0, OWN)])
            plsc.subcore_barrier()
            pltpu.sync_copy(garr, local_t)

            zero_killed()

            def e2body(j, _):
                p16 = edges_v[pl.ds(j * L, L)]
                d16 = p16 & 16383
                s16 = lax.shift_right_logical(p16, 14)
                ls = plsc.load_gather(local_t, [s16])
                plsc.store_scatter(killed_t, [d16], ones16, mask=ls > 0)
                return 0
            lax.fori_loop(0, EPT // L, e2body, 0)

            plsc.subcore_barrier()
            pltpu.sync_copy(killed_t, pub.at[t])
            plsc.subcore_barrier()
            merge_or(rank_own)

            anyv = zeros16

            def ubody(j, anyv):
                sl = pl.ds(j * L, L)
                newm = (mask_t[pl.ds(own0 + j * L, L)]
                        * (1 - local_own[sl])
                        * (1 - jnp.minimum(rank_own[sl], 1)))
                local_t[pl.ds(j * L, L)] = newm
                return anyv | newm
            anyv = lax.fori_loop(0, OWN // L, ubody, anyv)

            pltpu.sync_copy(local_t.at[pl.ds(0, OWN)], garr.at[pl.ds(own0, OWN)])
            fbuf[0, pl.ds(0, L)] = jnp.minimum(anyv, 1)
            pltpu.sync_copy(fbuf.at[0], gflags.at[t])
            plsc.subcore_barrier()
            pltpu.sync_copy(garr, mask_t)
            pltpu.sync_copy(gflags, fbuf)
            plsc.subcore_barrier()

            accv = zeros16
            for tt in range(NS):
                accv = accv | fbuf[tt, pl.ds(0, L)]
            return plsc.cummax(accv)[L - 1]

        lax.while_loop(lambda go: go > 0, loop_body, jnp.int32(1))

        pltpu.sync_copy(mis_own, mis_hbm.at[pl.ds(own0, OWN)])

        def rmbody(j, _):
            sl = pl.ds(j * L, L)
            rsl = rank_t[pl.ds(own0 + j * L, L)]
            rank_own[sl] = jnp.where(mis_own[sl] > 0, rsl, BIG)
            return 0
        lax.fori_loop(0, OWN // L, rmbody, 0)
        pltpu.sync_copy(rank_own, garr.at[pl.ds(own0, OWN)])
        plsc.subcore_barrier()
        pltpu.sync_copy(garr, mask_t)
        plsc.subcore_barrier()

        def cinit(j, _):
            killed_t[pl.ds(j * L, L)] = jnp.full((L,), BIG, i32)
            return 0
        lax.fori_loop(0, NPAD // L, cinit, 0)
        kbuf[pl.ds(0, L)] = jnp.full((L,), -1, i32)
        kbuf[pl.ds(2 * L, L)] = jnp.full((L,), -1, i32)
        vsbuf[pl.ds(2 * L, L)] = jnp.full((L,), BIG, i32)

        def cedge(j, _):
            p16 = edges_v[pl.ds(j * L, L)]
            d16 = p16 & 16383
            s16 = lax.shift_right_logical(p16, 14)
            rv = plsc.load_gather(mask_t, [s16])
            ks, vs = plsc.sort_key_val(d16, rv)
            kbuf[pl.ds(L, L)] = ks
            vsbuf[pl.ds(L, L)] = vs
            vc = vs
            for st in (1, 2, 4, 8):
                ksh = kbuf[pl.ds(L + st, L)]
                vsh = vsbuf[pl.ds(L + st, L)]
                vc = jnp.where(ks == ksh, jnp.minimum(vc, vsh), vc)
                vsbuf[pl.ds(L, L)] = vc
            kprev = kbuf[pl.ds(L - 1, L)]
            fo = ks != kprev
            old = plsc.load_gather(killed_t, [ks])
            plsc.store_scatter(killed_t, [ks], jnp.minimum(old, vc), mask=fo)
            return 0
        lax.fori_loop(0, EPT // L, cedge, 0)

        pltpu.sync_copy(killed_t, pub.at[t])
        plsc.subcore_barrier()
        pltpu.sync_copy(pub.at[:, pl.ds(own0, OWN)], mbuf)

        def cmerge(j, _):
            sl = pl.ds(j * L, L)
            acc = jnp.full((L,), BIG, i32)
            for tt in range(NS):
                acc = jnp.minimum(acc, mbuf[tt, sl])
            local_own[sl] = jnp.minimum(acc, rank_own[sl])
            return 0
        lax.fori_loop(0, OWN // L, cmerge, 0)
        pltpu.sync_copy(local_own, cand_hbm.at[pl.ds(own0, OWN)])

    return mis_k


def _kmis(score, src, dst, N, V, node_valid, edge_valid):
    s = score.reshape(-1)
    s_eff = jnp.where(node_valid, s, -jnp.inf)
    perm = jnp.argsort(-s_eff).astype(jnp.int32)
    ss = jnp.concatenate([src, dst])
    dd = jnp.concatenate([dst, src])
    em = jnp.concatenate([edge_valid, edge_valid])

    ssm = jnp.where(em, ss, N)
    ddm = jnp.where(em, dd, N)
    packed = ssm * jnp.int32(16384) + ddm
    misI, rankO, candO = _make_mis(N, ss.shape[0])(
        packed, perm, jnp.full((16,), V, jnp.int32))
    mis = misI[:N] > 0
    cand = candO[:N]
    cluster_node = perm[jnp.clip(cand, 0, N - 1)]
    Nc = jnp.sum(mis).astype(jnp.int32)
    new_id = jnp.where(mis, jnp.cumsum(mis.astype(jnp.int32)) - 1, 0)
    cluster = new_id[cluster_node]
    cu = cluster[src]
    cv = cluster[dst]
    keep = (cu != cv) & edge_valid
    SENT = jnp.int32(jnp.iinfo(jnp.int32).max)
    key = jnp.sort(jnp.where(keep, cu * Nc + cv, SENT))
    uniq = (key < SENT) & jnp.concatenate(
        [jnp.ones((1,), bool), key[1:] != key[:-1]])
    den = jnp.maximum(Nc, 1)
    new_src = jnp.where(uniq, key // den, 0).astype(jnp.int32)
    new_dst = jnp.where(uniq, key % den, 0).astype(jnp.int32)
    return mis, new_id, Nc, new_src, new_dst, uniq


def _dinv_of(deg):
    return jnp.where(deg > 0, lax.rsqrt(deg), 0.0)[:, None]


def kernel(x, edge_index, batch, W1, b1, ws1, bs1, W2, b2, ws2, bs2,
           W3, b3, Wl1, bl1, Wl2, bl2):
    src = edge_index[0]
    dst = edge_index[1]
    N = x.shape[0]
    E = src.shape[0]
    ones_n = jnp.ones((N,), bool)
    ones_e = jnp.ones((E,), bool)
    ones_col = jnp.ones((N, 1), jnp.float32)
    iota_n = jnp.arange(N, dtype=jnp.int32)

    deg1 = jnp.zeros((N,), jnp.float32).at[dst].add(1.0) + 1.0
    dinv1 = _dinv_of(deg1)
    hs1 = _mm_scale(x, W1, dinv1)
    acc1 = _aggregate(hs1, src, dst)
    h, s1 = _epilogue(acc1, hs1, dinv1, ones_col, b1, ws1)
    s1 = s1 + bs1

    mis1, nid1, Nc1, src1, dst1, ev2 = _kmis(s1, src, dst, N, jnp.int32(N),
                                             ones_n, ones_e)
    idx1 = jnp.where(mis1, nid1, N)
    val1 = h * s1
    h = jnp.zeros_like(val1).at[idx1].set(val1, mode="drop")
    nv2 = iota_n < Nc1
    nv2f = nv2.astype(jnp.float32)

    dstm2 = jnp.where(ev2, dst1, N)
    deg2 = (jnp.zeros((N,), jnp.float32)
            .at[dst1].add(jnp.where(ev2, 1.0, 0.0)) + nv2f)
    dinv2 = _dinv_of(deg2)
    hs2 = _mm_scale(h, W2, dinv2)
    acc2 = _aggregate(hs2, src1, dstm2)
    h, s2 = _epilogue(acc2, hs2, dinv2, nv2f[:, None], b2, ws2)
    s2 = s2 + bs2

    mis2, nid2, Nc2, src2, dst2, ev3 = _kmis(s2, src1, dst1, N, Nc1, nv2, ev2)
    idx2 = jnp.where(mis2, nid2, N)
    val2 = h * s2
    h = jnp.zeros_like(val2).at[idx2].set(val2, mode="drop")
    nv3 = iota_n < Nc2
    nv3f = nv3.astype(jnp.float32)

    dstm3 = jnp.where(ev3, dst2, N)
    deg3 = (jnp.zeros((N,), jnp.float32)
            .at[dst2].add(jnp.where(ev3, 1.0, 0.0)) + nv3f)
    dinv3 = _dinv_of(deg3)
    hs3 = _mm_scale(h, W3, dinv3)
    acc3 = _aggregate(hs3, src2, dstm3)
    h, _ = _epilogue(acc3, hs3, dinv3, nv3f[:, None], b3,
                     jnp.zeros((W3.shape[1], 1), jnp.float32))

    return _head(h, nv3f, Wl1, bl1, Wl2, bl2)


# speedup vs baseline: 32.6683x; 2.1558x over previous
import functools

import jax
import jax.numpy as jnp
from jax import lax
from jax.experimental import pallas as pl
from jax.experimental.pallas import tpu as pltpu
from jax.experimental.pallas import tpu_sc as plsc

NC = 2
NS = 16
L = 16


def _mm_scale_body(x_ref, w_ref, dinv_ref, o_ref):
    hw = jnp.dot(x_ref[...], w_ref[...], preferred_element_type=jnp.float32)
    o_ref[...] = dinv_ref[...] * hw


def _mm_scale(x, W, dinv, block_m=2000):
    M, K = x.shape
    _, N = W.shape
    return pl.pallas_call(
        _mm_scale_body,
        grid=(M // block_m,),
        in_specs=[
            pl.BlockSpec((block_m, K), lambda i: (i, 0)),
            pl.BlockSpec((K, N), lambda i: (0, 0)),
            pl.BlockSpec((block_m, 1), lambda i: (i, 0)),
        ],
        out_specs=pl.BlockSpec((block_m, N), lambda i: (i, 0)),
        out_shape=jax.ShapeDtypeStruct((M, N), jnp.float32),
    )(x, W, dinv)


def _epi_body(acc_ref, hs_ref, dinv_ref, nv_ref, b_ref, ws_ref, h_ref, s_ref):
    h = dinv_ref[...] * (acc_ref[...] + hs_ref[...]) + b_ref[...]
    h = jnp.maximum(h, 0.0) * nv_ref[...]
    h_ref[...] = h
    s_ref[...] = jnp.dot(h, ws_ref[...], preferred_element_type=jnp.float32)


def _epilogue(acc, hs, dinv, nv, b, ws, block_m=2000):
    M, N = acc.shape
    return pl.pallas_call(
        _epi_body,
        grid=(M // block_m,),
        in_specs=[
            pl.BlockSpec((block_m, N), lambda i: (i, 0)),
            pl.BlockSpec((block_m, N), lambda i: (i, 0)),
            pl.BlockSpec((block_m, 1), lambda i: (i, 0)),
            pl.BlockSpec((block_m, 1), lambda i: (i, 0)),
            pl.BlockSpec((1, N), lambda i: (0, 0)),
            pl.BlockSpec((N, 1), lambda i: (0, 0)),
        ],
        out_specs=(pl.BlockSpec((block_m, N), lambda i: (i, 0)),
                   pl.BlockSpec((block_m, 1), lambda i: (i, 0))),
        out_shape=(jax.ShapeDtypeStruct((M, N), jnp.float32),
                   jax.ShapeDtypeStruct((M, 1), jnp.float32)),
    )(acc, hs, dinv, nv, b.reshape(1, -1), ws)


def _head_body(h_ref, nv_ref, wl1_ref, bl1_ref, wl2_ref, bl2_ref, o_ref):
    h = h_ref[...]
    nv = nv_ref[...]
    gmax = jnp.max(jnp.where(nv > 0, h, -jnp.inf), axis=0, keepdims=True)
    gsum = jnp.sum(h, axis=0, keepdims=True)
    cnt = jnp.maximum(jnp.sum(nv), 1.0)
    g = jnp.concatenate([gmax, gsum / cnt], axis=1)
    z = jnp.maximum(jnp.dot(g, wl1_ref[...], preferred_element_type=jnp.float32)
                    + bl1_ref[...], 0.0)
    logits = jnp.dot(z, wl2_ref[...], preferred_element_type=jnp.float32) + bl2_ref[...]
    o_ref[...] = jax.nn.log_softmax(logits, axis=-1)


def _head(h, nv_f32, Wl1, bl1, Wl2, bl2):
    M, _ = h.shape
    return pl.pallas_call(
        _head_body,
        out_shape=jax.ShapeDtypeStruct((1, Wl2.shape[1]), jnp.float32),
    )(h, nv_f32.reshape(M, 1), Wl1, bl1.reshape(1, -1), Wl2, bl2.reshape(1, -1))


@functools.cache
def _make_agg(N, E, F):
    CH = {128: 128, 256: 64, 512: 32}[F]
    SEG = 8000 if F <= 128 else 4000
    NP = 2 if F >= 512 else 1
    P = NC * NS * NP
    R = ((N + P - 1) // P + 7) // 8 * 8
    CAP = SEG + CH
    NSEG = (E + SEG - 1) // SEG
    assert E % SEG == 0
    mesh = plsc.VectorSubcoreMesh(core_axis_name="c", subcore_axis_name="s")

    @functools.partial(
        pl.kernel,
        out_type=jax.ShapeDtypeStruct((P * R, F), jnp.float32),
        mesh=mesh,
        compiler_params=pltpu.CompilerParams(needs_layout_passes=False),
        scratch_types=[
            pltpu.VMEM((SEG,), jnp.int32),
            pltpu.VMEM((SEG,), jnp.int32),
            pltpu.VMEM((CAP,), jnp.int32),
            pltpu.VMEM((CAP,), jnp.int32),
            pltpu.VMEM((CH,), jnp.int32),
            pltpu.VMEM((CH, F), jnp.float32),
            pltpu.VMEM((R + 8, F), jnp.float32),
            pltpu.SemaphoreType.DMA,
        ],
    )
    def agg(hs_hbm, src_hbm, dst_hbm, zrows_hbm, out_hbm,
            src_v, dst_v, csrc, cloc, gidx, gbuf, acc, gsem):
        c = lax.axis_index("c")
        s = lax.axis_index("s")
        for q in range(NP):
            pt = (c * NS + s) * NP + q
            shift = {32: 5, 64: 6}[P]

            off = 0
            while off < R:
                n = min(128, R - off)
                pltpu.sync_copy(zrows_hbm.at[pl.ds(0, n)],
                                acc.at[pl.ds(off, n)])
                off += n

            def seg_body(g, _):
                pltpu.sync_copy(src_hbm.at[pl.ds(g * SEG, SEG)], src_v)
                pltpu.sync_copy(dst_hbm.at[pl.ds(g * SEG, SEG)], dst_v)

                def cbody(i, m):
                    s16 = src_v[pl.ds(i * L, L)]
                    d16 = dst_v[pl.ds(i * L, L)]
                    g16 = lax.shift_right_logical(d16, 3)
                    inb = ((g16 & (P - 1)) == pt) & (d16 < N)
                    dloc = (lax.shift_left(
                        lax.shift_right_logical(g16, shift), 3)
                        | (d16 & 7))
                    inc = plsc.cumsum(inb.astype(jnp.int32))
                    pos = m + inc - inb.astype(jnp.int32)
                    plsc.store_scatter(csrc, [pos], s16, mask=inb)
                    plsc.store_scatter(cloc, [pos], dloc, mask=inb)
                    return m + inc[L - 1]

                m = lax.fori_loop(0, SEG // L, cbody, jnp.int32(0))

                pad16 = (c * NS + s) * L + lax.iota(jnp.int32, L)
                for t in range(CH // L):
                    csrc[pl.ds(m + t * L, L)] = pad16
                    cloc[pl.ds(m + t * L, L)] = jnp.full((L,), R, jnp.int32)

                nch = (m + CH - 1) // CH

                def chunk_body(j, _):
                    base = j * CH
                    for k in range(CH // L):
                        gidx[pl.ds(k * L, L)] = csrc[pl.ds(base + k * L, L)]
                    pltpu.async_copy(hs_hbm.at[gidx], gbuf, gsem).wait()
                    nv16 = (jnp.minimum(CH, m - base) + L - 1) // L

                    def row_body(i, _):
                        dl16 = cloc[pl.ds(base + i * L, L)]
                        for t in range(L):
                            dl = dl16[t]
                            for k in range(F // L):
                                sl = pl.ds(k * L, L)
                                acc[dl, sl] = acc[dl, sl] + gbuf[i * L + t, sl]
                        return 0

                    lax.fori_loop(0, nv16, row_body, 0)
                    return 0

                lax.fori_loop(0, nch, chunk_body, 0)
                return 0

            lax.fori_loop(0, NSEG, seg_body, 0)

            off = 0
            while off < R:
                n = min(256, R - off)
                pltpu.sync_copy(acc.at[pl.ds(off, n)],
                                out_hbm.at[pl.ds(pt * R + off, n)])
                off += n

    return agg, P, R


def _aggregate(hs, src, dst_masked):
    N, F = hs.shape
    E = src.shape[0]
    zrows = jnp.zeros((128, F), jnp.float32)
    agg, P, R = _make_agg(N, E, F)
    outp = agg(hs, src, dst_masked, zrows)
    outp = outp.reshape(P, R // 8, 8, F).transpose(1, 0, 2, 3).reshape(-1, F)
    return lax.slice_in_dim(outp, 0, N)


@functools.cache
def _make_mis(N, E2):
    NPAD = 10240
    OWN = NPAD // NS
    EPT = E2 // NS
    BIG = N
    mesh = plsc.VectorSubcoreMesh(core_axis_name="c", subcore_axis_name="s",
                                  num_cores=1)
    i32 = jnp.int32

    @functools.partial(
        pl.kernel,
        out_type=(jax.ShapeDtypeStruct((NPAD,), i32),
                  jax.ShapeDtypeStruct((NPAD,), i32),
                  jax.ShapeDtypeStruct((NPAD,), i32),
                  jax.ShapeDtypeStruct((NPAD,), i32),
                  jax.ShapeDtypeStruct((E2 // 2,), i32)),
        mesh=mesh,
        compiler_params=pltpu.CompilerParams(needs_layout_passes=False),
        scratch_types=[
            pltpu.VMEM((EPT,), i32),
            pltpu.VMEM((NPAD,), i32),
            pltpu.VMEM((NPAD,), i32),
            pltpu.VMEM((NPAD,), i32),
            pltpu.VMEM((NPAD,), i32),
            pltpu.VMEM((OWN,), i32),
            pltpu.VMEM((OWN,), i32),
            pltpu.VMEM((OWN,), i32),
            pltpu.VMEM((NS, OWN), i32),
            pltpu.VMEM((NS, L), i32),
            pltpu.VMEM((L,), i32),
            pltpu.VMEM((48,), i32),
            pltpu.VMEM((48,), i32),
            pltpu.VMEM((OWN,), i32),
            pltpu.VMEM_SHARED((NS, NPAD), i32),
            pltpu.VMEM_SHARED((NPAD,), i32),
            pltpu.VMEM_SHARED((NS, L), i32),
        ],
    )
    def mis_k(packed_hbm, perm_hbm, v_hbm,
              mis_hbm, rank_hbm, cand_hbm, nid_hbm, keys_hbm,
              edges_v, rank_t, mask_t, killed_t, local_t,
              rank_own, local_own, mis_own, mbuf, fbuf, vbuf, kbuf, vsbuf,
              nid_own, pub, garr, gflags):
        t = lax.axis_index("s")
        own0 = t * OWN
        iota = lax.iota(i32, L)
        ones16 = jnp.ones((L,), i32)
        zeros16 = jnp.zeros((L,), i32)

        pltpu.sync_copy(packed_hbm.at[pl.ds(t * EPT, EPT)], edges_v)
        pltpu.sync_copy(v_hbm, vbuf)
        V = vbuf[pl.ds(0, L)][0]

        def ibody(j, _):
            idx16 = j * L + iota
            mask_t[pl.ds(j * L, L)] = (idx16 < V).astype(i32)
            return 0
        lax.fori_loop(0, NPAD // L, ibody, 0)

        def i2body(j, _):
            sl = pl.ds(j * L, L)
            rank_own[sl] = jnp.full((L,), BIG, i32)
            mis_own[sl] = zeros16
            return 0
        lax.fori_loop(0, OWN // L, i2body, 0)

        pltpu.sync_copy(perm_hbm, local_t.at[pl.ds(0, N)])

        def rbody(j, _):
            p16 = local_t[pl.ds(j * L, L)]
            inown = (p16 >= own0) & (p16 < own0 + OWN)
            plsc.store_scatter(rank_own, [p16 - own0], j * L + iota, mask=inown)
            return 0
        lax.fori_loop(0, N // L, rbody, 0)

        pltpu.sync_copy(rank_own, garr.at[pl.ds(own0, OWN)])
        plsc.subcore_barrier()
        pltpu.sync_copy(garr, rank_t)
        pltpu.sync_copy(rank_t.at[pl.ds(own0, OWN)],
                        rank_hbm.at[pl.ds(own0, OWN)])
        plsc.subcore_barrier()

        def zero_killed():
            def zbody(j, _):
                killed_t[pl.ds(j * L, L)] = zeros16
                return 0
            lax.fori_loop(0, NPAD // L, zbody, 0)

        def merge_or(dst_own):
            pltpu.sync_copy(pub.at[:, pl.ds(own0, OWN)], mbuf)

            def obody(j, _):
                sl = pl.ds(j * L, L)
                acc = zeros16
                for tt in range(NS):
                    acc = acc | mbuf[tt, sl]
                dst_own[sl] = acc
                return 0
            lax.fori_loop(0, OWN // L, obody, 0)

        def loop_body(go):
            zero_killed()

            def e1body(j, _):
                p16 = edges_v[pl.ds(j * L, L)]
                d16 = p16 & 16383
                s16 = lax.shift_right_logical(p16, 14)
                rs = plsc.load_gather(rank_t, [s16])
                rd = plsc.load_gather(rank_t, [d16])
                ms = plsc.load_gather(mask_t, [s16])
                ind = (ms > 0) & (rs < rd)
                plsc.store_scatter(killed_t, [d16], ones16, mask=ind)
                return 0
            lax.fori_loop(0, EPT // L, e1body, 0)

            pltpu.sync_copy(killed_t, pub.at[t])
            plsc.subcore_barrier()
            merge_or(local_own)

            def lbody(j, _):
                sl = pl.ds(j * L, L)
                loc = jnp.where(mask_t[pl.ds(own0 + j * L, L)] > 0,
                                1 - jnp.minimum(local_own[sl], 1), 0)
                local_own[sl] = loc
                mis_own[sl] = mis_own[sl] | loc
                return 0
            lax.fori_loop(0, OWN // L, lbody, 0)

            plsc.subcore_barrier()
            pltpu.sync_copy(local_own, garr.at[pl.ds(own0, OWN)])
            plsc.subcore_barrier()
            pltpu.sync_copy(garr, local_t)

            zero_killed()

            def e2body(j, _):
                p16 = edges_v[pl.ds(j * L, L)]
                d16 = p16 & 16383
                s16 = lax.shift_right_logical(p16, 14)
                ls = plsc.load_gather(local_t, [s16])
                plsc.store_scatter(killed_t, [d16], ones16, mask=ls > 0)
                return 0
            lax.fori_loop(0, EPT // L, e2body, 0)

            plsc.subcore_barrier()
            pltpu.sync_copy(killed_t, pub.at[t])
            plsc.subcore_barrier()
            merge_or(rank_own)

            anyv = zeros16

            def ubody(j, anyv):
                sl = pl.ds(j * L, L)
                newm = (mask_t[pl.ds(own0 + j * L, L)]
                        * (1 - local_own[sl])
                        * (1 - jnp.minimum(rank_own[sl], 1)))
                local_t[pl.ds(j * L, L)] = newm
                return anyv | newm
            anyv = lax.fori_loop(0, OWN // L, ubody, anyv)

            pltpu.sync_copy(local_t.at[pl.ds(0, OWN)], garr.at[pl.ds(own0, OWN)])
            fbuf[0, pl.ds(0, L)] = jnp.minimum(anyv, 1)
            pltpu.sync_copy(fbuf.at[0], gflags.at[t])
            plsc.subcore_barrier()
            pltpu.sync_copy(garr, mask_t)
            pltpu.sync_copy(gflags, fbuf)
            plsc.subcore_barrier()

            accv = zeros16
            for tt in range(NS):
                accv = accv | fbuf[tt, pl.ds(0, L)]
            return plsc.cummax(accv)[L - 1]

        lax.while_loop(lambda go: go > 0, loop_body, jnp.int32(1))

        pltpu.sync_copy(mis_own, mis_hbm.at[pl.ds(own0, OWN)])

        def rmbody(j, _):
            sl = pl.ds(j * L, L)
            rsl = rank_t[pl.ds(own0 + j * L, L)]
            rank_own[sl] = jnp.where(mis_own[sl] > 0, rsl, BIG)
            return 0
        lax.fori_loop(0, OWN // L, rmbody, 0)
        pltpu.sync_copy(rank_own, garr.at[pl.ds(own0, OWN)])
        plsc.subcore_barrier()
        pltpu.sync_copy(garr, mask_t)
        plsc.subcore_barrier()

        def cinit(j, _):
            killed_t[pl.ds(j * L, L)] = jnp.full((L,), BIG, i32)
            return 0
        lax.fori_loop(0, NPAD // L, cinit, 0)
        kbuf[pl.ds(0, L)] = jnp.full((L,), -1, i32)
        kbuf[pl.ds(2 * L, L)] = jnp.full((L,), -1, i32)
        vsbuf[pl.ds(2 * L, L)] = jnp.full((L,), BIG, i32)

        def cedge(j, _):
            p16 = edges_v[pl.ds(j * L, L)]
            d16 = p16 & 16383
            s16 = lax.shift_right_logical(p16, 14)
            rv = plsc.load_gather(mask_t, [s16])
            ks, vs = plsc.sort_key_val(d16, rv)
            kbuf[pl.ds(L, L)] = ks
            vsbuf[pl.ds(L, L)] = vs
            vc = vs
            for st in (1, 2, 4, 8):
                ksh = kbuf[pl.ds(L + st, L)]
                vsh = vsbuf[pl.ds(L + st, L)]
                vc = jnp.where(ks == ksh, jnp.minimum(vc, vsh), vc)
                vsbuf[pl.ds(L, L)] = vc
            kprev = kbuf[pl.ds(L - 1, L)]
            fo = ks != kprev
            old = plsc.load_gather(killed_t, [ks])
            plsc.store_scatter(killed_t, [ks], jnp.minimum(old, vc), mask=fo)
            return 0
        lax.fori_loop(0, EPT // L, cedge, 0)

        pltpu.sync_copy(killed_t, pub.at[t])
        plsc.subcore_barrier()
        pltpu.sync_copy(pub.at[:, pl.ds(own0, OWN)], mbuf)

        def cmerge(j, _):
            sl = pl.ds(j * L, L)
            acc = jnp.full((L,), BIG, i32)
            for tt in range(NS):
                acc = jnp.minimum(acc, mbuf[tt, sl])
            local_own[sl] = jnp.minimum(acc, rank_own[sl])
            return 0
        lax.fori_loop(0, OWN // L, cmerge, 0)
        pltpu.sync_copy(local_own, cand_hbm.at[pl.ds(own0, OWN)])

        def cnt_body(j, run):
            cs = plsc.cumsum(mis_own[pl.ds(j * L, L)])
            return run + cs[L - 1]
        cnt = lax.fori_loop(0, OWN // L, cnt_body, jnp.int32(0))
        fbuf[0, pl.ds(0, L)] = jnp.full((L,), cnt, i32)
        pltpu.sync_copy(fbuf.at[0], gflags.at[t])
        plsc.subcore_barrier()
        pltpu.sync_copy(gflags, fbuf)
        base = jnp.int32(0)
        nc = jnp.int32(0)
        for tt in range(NS):
            v = fbuf[tt, pl.ds(0, L)][0]
            base = base + jnp.where(tt < t, v, 0)
            nc = nc + v

        def nid_body(j, run):
            m16 = mis_own[pl.ds(j * L, L)]
            cs = plsc.cumsum(m16)
            nid_own[pl.ds(j * L, L)] = jnp.where(
                m16 > 0, base + run + cs - 1, 0)
            return run + cs[L - 1]
        lax.fori_loop(0, OWN // L, nid_body, jnp.int32(0))
        pltpu.sync_copy(nid_own, nid_hbm.at[pl.ds(own0, OWN)])
        pltpu.sync_copy(nid_own, garr.at[pl.ds(own0, OWN)])
        plsc.subcore_barrier()
        pltpu.sync_copy(garr, killed_t)
        pltpu.sync_copy(perm_hbm, local_t.at[pl.ds(0, N)])

        def cn_body(j, _):
            sl = pl.ds(j * L, L)
            cl = jnp.clip(local_own[sl], 0, N - 1)
            rank_own[sl] = plsc.load_gather(local_t, [cl])
            return 0
        lax.fori_loop(0, OWN // L, cn_body, 0)

        def clu_body(j, _):
            sl = pl.ds(j * L, L)
            nid_own[sl] = plsc.load_gather(killed_t, [rank_own[sl]])
            return 0
        lax.fori_loop(0, OWN // L, clu_body, 0)
        plsc.subcore_barrier()
        pltpu.sync_copy(nid_own, garr.at[pl.ds(own0, OWN)])
        plsc.subcore_barrier()
        pltpu.sync_copy(garr, mask_t)

        SENT = jnp.int32(2147483647)

        @pl.when(t < NS // 2)
        def _keys():
            def key_body(j, _):
                p16 = edges_v[pl.ds(j * L, L)]
                d16 = p16 & 16383
                s16 = lax.shift_right_logical(p16, 14)
                ku = plsc.load_gather(mask_t, [s16])
                kv = plsc.load_gather(mask_t, [d16])
                keep = (d16 < N) & (ku != kv)
                edges_v[pl.ds(j * L, L)] = jnp.where(keep, ku * nc + kv, SENT)
                return 0
            lax.fori_loop(0, EPT // L, key_body, 0)
            pltpu.sync_copy(edges_v, keys_hbm.at[pl.ds(t * EPT, EPT)])

    return mis_k


def _kmis(score, src, dst, N, V, node_valid, edge_valid):
    s = score.reshape(-1)
    s_eff = jnp.where(node_valid, s, -jnp.inf)
    perm = jnp.argsort(-s_eff).astype(jnp.int32)
    ss = jnp.concatenate([src, dst])
    dd = jnp.concatenate([dst, src])
    em = jnp.concatenate([edge_valid, edge_valid])

    ssm = jnp.where(em, ss, N)
    ddm = jnp.where(em, dd, N)
    packed = ssm * jnp.int32(16384) + ddm
    misI, rankO, candO, nidO, keysO = _make_mis(N, ss.shape[0])(
        packed, perm, jnp.full((16,), V, jnp.int32))
    mis = misI[:N] > 0
    Nc = jnp.sum(misI[:N]).astype(jnp.int32)
    new_id = nidO[:N]
    SENT = jnp.int32(jnp.iinfo(jnp.int32).max)
    key = jnp.sort(keysO)
    uniq = (key < SENT) & jnp.concatenate(
        [jnp.ones((1,), bool), key[1:] != key[:-1]])
    den = jnp.maximum(Nc, 1)
    new_src = jnp.where(uniq, key // den, 0).astype(jnp.int32)
    new_dst = jnp.where(uniq, key % den, 0).astype(jnp.int32)
    return mis, new_id, Nc, new_src, new_dst, uniq


def _dinv_of(deg):
    return jnp.where(deg > 0, lax.rsqrt(deg), 0.0)[:, None]


def kernel(x, edge_index, batch, W1, b1, ws1, bs1, W2, b2, ws2, bs2,
           W3, b3, Wl1, bl1, Wl2, bl2):
    src = edge_index[0]
    dst = edge_index[1]
    N = x.shape[0]
    E = src.shape[0]
    ones_n = jnp.ones((N,), bool)
    ones_e = jnp.ones((E,), bool)
    ones_col = jnp.ones((N, 1), jnp.float32)
    iota_n = jnp.arange(N, dtype=jnp.int32)

    deg1 = jnp.zeros((N,), jnp.float32).at[dst].add(1.0) + 1.0
    dinv1 = _dinv_of(deg1)
    hs1 = _mm_scale(x, W1, dinv1)
    acc1 = _aggregate(hs1, src, dst)
    h, s1 = _epilogue(acc1, hs1, dinv1, ones_col, b1, ws1)
    s1 = s1 + bs1

    mis1, nid1, Nc1, src1, dst1, ev2 = _kmis(s1, src, dst, N, jnp.int32(N),
                                             ones_n, ones_e)
    idx1 = jnp.where(mis1, nid1, N)
    val1 = h * s1
    h = jnp.zeros_like(val1).at[idx1].set(val1, mode="drop")
    nv2 = iota_n < Nc1
    nv2f = nv2.astype(jnp.float32)

    dstm2 = jnp.where(ev2, dst1, N)
    deg2 = (jnp.zeros((N,), jnp.float32)
            .at[dst1].add(jnp.where(ev2, 1.0, 0.0)) + nv2f)
    dinv2 = _dinv_of(deg2)
    hs2 = _mm_scale(h, W2, dinv2)
    acc2 = _aggregate(hs2, src1, dstm2)
    h, s2 = _epilogue(acc2, hs2, dinv2, nv2f[:, None], b2, ws2)
    s2 = s2 + bs2

    mis2, nid2, Nc2, src2, dst2, ev3 = _kmis(s2, src1, dst1, N, Nc1, nv2, ev2)
    idx2 = jnp.where(mis2, nid2, N)
    val2 = h * s2
    h = jnp.zeros_like(val2).at[idx2].set(val2, mode="drop")
    nv3 = iota_n < Nc2
    nv3f = nv3.astype(jnp.float32)

    dstm3 = jnp.where(ev3, dst2, N)
    deg3 = (jnp.zeros((N,), jnp.float32)
            .at[dst2].add(jnp.where(ev3, 1.0, 0.0)) + nv3f)
    dinv3 = _dinv_of(deg3)
    hs3 = _mm_scale(h, W3, dinv3)
    acc3 = _aggregate(hs3, src2, dstm3)
    h, _ = _epilogue(acc3, hs3, dinv3, nv3f[:, None], b3,
                     jnp.zeros((W3.shape[1], 1), jnp.float32))

    return _head(h, nv3f, Wl1, bl1, Wl2, bl2)
